# Initial kernel scaffold; baseline (speedup 1.0000x reference)
import functools

import jax
import jax.numpy as jnp
from jax import lax
from jax.experimental import pallas as pl
from jax.experimental.pallas import tpu as pltpu
from jax.experimental.pallas import tpu_sc as plsc

KNN = 16
B, N, N2 = 8, 2048, 2048
C = 64
TI = 256
TT = 4096
EPS = 1e-5
NEG_SLOPE = 0.1

SC_WORKERS = 32
TOTAL_ROWS = B * N * KNN
ROWS_PER_W = TOTAL_ROWS // SC_WORKERS
SC_CHUNK = 128
SC_NCHUNK = ROWS_PER_W // SC_CHUNK


def _knn_proj_body(xyz1_ref, xyz2_ref, feat1_ref, feat2_ref, w0_ref, b0_ref,
                   idx_ref, base_ref, pc_ref, s_ref):
  b = pl.program_id(0)
  it = pl.program_id(1)
  x1 = xyz1_ref[0]
  x2 = xyz2_ref[0]
  w0 = w0_ref[...]

  n2 = jnp.sum(x2 * x2, axis=0, keepdims=True)
  g = lax.dot_general(x1, x2, (((0,), (0,)), ((), ())),
                      preferred_element_type=jnp.float32)
  s_ref[...] = n2 - 2.0 * g

  iota = lax.broadcasted_iota(jnp.int32, (8, N2), 1)

  def blk(j, carry):
    sb = s_ref[pl.ds(j * 8, 8), :]
    cols = []
    for r in range(KNN):
      m = jnp.min(sb, axis=1, keepdims=True)
      am = jnp.min(jnp.where(sb == m, iota, N2), axis=1,
                   keepdims=True)
      cols.append(am)
      if r < KNN - 1:
        sb = jnp.where(iota == am, jnp.inf, sb)
    idxb = jnp.concatenate(cols, axis=1)
    idx_ref[0, pl.ds(j * 8, 8), :] = idxb + b * N2
    return carry

  lax.fori_loop(0, TI // 8, blk, 0)

  f1 = feat1_ref[0]
  bt = (lax.dot_general(f1, w0[:, :C], (((0,), (1,)), ((), ())),
                        preferred_element_type=jnp.float32)
        - lax.dot_general(x1, w0[:, 2 * C:], (((0,), (1,)), ((), ())),
                          preferred_element_type=jnp.float32)
        + b0_ref[...])
  base_ref[0] = bt

  @pl.when(it == 0)
  def _():
    f2 = feat2_ref[0]
    pcv = (lax.dot_general(f2, w0[:, C:2 * C], (((0,), (1,)), ((), ())),
                           preferred_element_type=jnp.float32)
           + lax.dot_general(x2, w0[:, 2 * C:], (((0,), (1,)), ((), ())),
                             preferred_element_type=jnp.float32))
    pc_ref[0] = pcv


def _knn_proj_call(xyz1, xyz2, feat1, feat2, w0, b0_2d):
  return pl.pallas_call(
      _knn_proj_body,
      grid=(B, N // TI),
      in_specs=[
          pl.BlockSpec((1, 3, TI), lambda b, it: (b, 0, it)),
          pl.BlockSpec((1, 3, N2), lambda b, it: (b, 0, 0)),
          pl.BlockSpec((1, C, TI), lambda b, it: (b, 0, it)),
          pl.BlockSpec((1, C, N2), lambda b, it: (b, 0, 0)),
          pl.BlockSpec((C, 131), lambda b, it: (0, 0)),
          pl.BlockSpec((1, C), lambda b, it: (0, 0)),
      ],
      out_specs=[
          pl.BlockSpec((1, TI, KNN), lambda b, it: (b, it, 0)),
          pl.BlockSpec((1, TI, C), lambda b, it: (b, it, 0)),
          pl.BlockSpec((1, N2, C), lambda b, it: (b, 0, 0)),
      ],
      out_shape=[
          jax.ShapeDtypeStruct((B, N, KNN), jnp.int32),
          jax.ShapeDtypeStruct((B, N, C), jnp.float32),
          jax.ShapeDtypeStruct((B, N2, C), jnp.float32),
      ],
      scratch_shapes=[pltpu.VMEM((TI, N2), jnp.float32)],
      compiler_params=pltpu.CompilerParams(
          dimension_semantics=("arbitrary", "arbitrary")),
  )(xyz1, xyz2, feat1, feat2, w0, b0_2d)


@functools.partial(
    pl.kernel,
    out_type=jax.ShapeDtypeStruct((TOTAL_ROWS, C), jnp.float32),
    mesh=plsc.VectorSubcoreMesh(core_axis_name="c", subcore_axis_name="s"),
    scratch_types=[
        pltpu.VMEM((SC_CHUNK,), jnp.int32),
        pltpu.VMEM((SC_CHUNK, C), jnp.float32),
        pltpu.SemaphoreType.DMA,
    ],
)
def _sc_gather(table_hbm, idx_hbm, out_hbm, idx_v, rows_v, sem):
  wid = lax.axis_index("s") * 2 + lax.axis_index("c")
  base = wid * ROWS_PER_W

  def chunk(ci, carry):
    off = base + ci * SC_CHUNK
    pltpu.sync_copy(idx_hbm.at[pl.ds(off, SC_CHUNK)], idx_v)
    pltpu.async_copy(table_hbm.at[idx_v], rows_v, sem).wait()
    pltpu.sync_copy(rows_v, out_hbm.at[pl.ds(off, SC_CHUNK)])
    return carry

  lax.fori_loop(0, SC_NCHUNK, chunk, 0)


def _group_mat():
  i = lax.broadcasted_iota(jnp.int32, (C, C), 0)
  j = lax.broadcasted_iota(jnp.int32, (C, C), 1)
  return ((i // 16) == (j // 16)).astype(jnp.float32)


def _mlp_body(g_ref, base_ref, w1_ref, b1_ref, g0_ref, beta0_ref,
              g1_ref, beta1_ref, out_ref, z_ref):
  nt = (N * KNN) // TT
  qt = TT // KNN
  gm = _group_mat()
  cnt = 16.0 * N * KNN

  def tile_y(t):
    gt = g_ref[0, pl.ds(t * TT, TT), :]
    bt = base_ref[0, pl.ds(t * qt, qt), :]
    y = gt.reshape(qt, KNN, C) + bt[:, None, :]
    return y.reshape(TT, C)

  def p1(t, carry):
    s, q = carry
    y = tile_y(t)
    return (s + jnp.sum(y, axis=0, keepdims=True),
            q + jnp.sum(y * y, axis=0, keepdims=True))

  z1 = jnp.zeros((1, C), jnp.float32)
  s0, q0 = lax.fori_loop(0, nt, p1, (z1, z1))
  mean0 = jnp.dot(s0, gm, preferred_element_type=jnp.float32) / cnt
  var0 = jnp.dot(q0, gm, preferred_element_type=jnp.float32) / cnt - mean0 * mean0
  inv0 = lax.rsqrt(var0 + EPS)
  sc0 = inv0 * g0_ref[...]
  sh0 = beta0_ref[...] - mean0 * sc0

  w1 = w1_ref[...]
  b1 = b1_ref[...]

  def p2(t, carry):
    s, q = carry
    ya = tile_y(t) * sc0 + sh0
    ya = jnp.where(ya >= 0, ya, NEG_SLOPE * ya)
    z = lax.dot_general(ya, w1, (((1,), (1,)), ((), ())),
                        preferred_element_type=jnp.float32) + b1
    z_ref[pl.ds(t * TT, TT), :] = z
    return (s + jnp.sum(z, axis=0, keepdims=True),
            q + jnp.sum(z * z, axis=0, keepdims=True))

  s1, q1 = lax.fori_loop(0, nt, p2, (z1, z1))
  mean1 = jnp.dot(s1, gm, preferred_element_type=jnp.float32) / cnt
  var1 = jnp.dot(q1, gm, preferred_element_type=jnp.float32) / cnt - mean1 * mean1
  inv1 = lax.rsqrt(var1 + EPS)
  sc1 = inv1 * g1_ref[...]
  sh1 = beta1_ref[...] - mean1 * sc1

  def p3(t, carry):
    z = z_ref[pl.ds(t * TT, TT), :]
    za = z * sc1 + sh1
    za = jnp.where(za >= 0, za, NEG_SLOPE * za)
    zm = jnp.max(za.reshape(qt, KNN, C), axis=1)
    out_ref[0, :, pl.ds(t * qt, qt)] = zm.T
    return carry

  lax.fori_loop(0, nt, p3, 0)


def _mlp_call(g, base, w1, b1_2d, g0_2d, beta0_2d, g1_2d, beta1_2d):
  return pl.pallas_call(
      _mlp_body,
      grid=(B,),
      in_specs=[
          pl.BlockSpec((1, N * KNN, C), lambda b: (b, 0, 0)),
          pl.BlockSpec((1, N, C), lambda b: (b, 0, 0)),
          pl.BlockSpec((C, C), lambda b: (0, 0)),
          pl.BlockSpec((1, C), lambda b: (0, 0)),
          pl.BlockSpec((1, C), lambda b: (0, 0)),
          pl.BlockSpec((1, C), lambda b: (0, 0)),
          pl.BlockSpec((1, C), lambda b: (0, 0)),
          pl.BlockSpec((1, C), lambda b: (0, 0)),
      ],
      out_specs=pl.BlockSpec((1, C, N), lambda b: (b, 0, 0)),
      out_shape=jax.ShapeDtypeStruct((B, C, N), jnp.float32),
      scratch_shapes=[pltpu.VMEM((N * KNN, C), jnp.float32)],
      compiler_params=pltpu.CompilerParams(
          dimension_semantics=("arbitrary",)),
  )(g, base, w1, b1_2d, g0_2d, beta0_2d, g1_2d, beta1_2d)


def kernel(xyz1, xyz2, feat1, feat2, W0, b0, g0, beta0, W1, b1, g1, beta1):
  idx, base, pc = _knn_proj_call(xyz1, xyz2, feat1, feat2, W0,
                                 b0.reshape(1, C))
  gathered = _sc_gather(pc.reshape(B * N2, C), idx.reshape(TOTAL_ROWS))
  return _mlp_call(gathered.reshape(B, N * KNN, C), base, W1,
                   b1.reshape(1, C), g0.reshape(1, C), beta0.reshape(1, C),
                   g1.reshape(1, C), beta1.reshape(1, C))


# speedup vs baseline: 1.6053x; 1.6053x over previous
import functools

import jax
import jax.numpy as jnp
from jax import lax
from jax.experimental import pallas as pl
from jax.experimental.pallas import tpu as pltpu
from jax.experimental.pallas import tpu_sc as plsc

KNN = 16
B, N, N2 = 8, 2048, 2048
C = 64
TI = 256
TT = 4096
EPS = 1e-5
NEG_SLOPE = 0.1

SC_WORKERS = 32
TOTAL_ROWS = B * N * KNN
ROWS_PER_W = TOTAL_ROWS // SC_WORKERS
SC_CHUNK = 128
SC_NCHUNK = ROWS_PER_W // SC_CHUNK


def _knn_proj_body(xyz1_ref, xyz2_ref, feat1_ref, feat2_ref, w0_ref, b0_ref,
                   idx_ref, base_ref, pc_ref, s_ref):
  b = pl.program_id(0)
  it = pl.program_id(1)
  x1 = xyz1_ref[0]
  x2 = xyz2_ref[0]
  w0 = w0_ref[...]

  n2 = jnp.sum(x2 * x2, axis=0, keepdims=True)
  g = lax.dot_general(x1, x2, (((0,), (0,)), ((), ())),
                      preferred_element_type=jnp.float32,
                      precision=lax.Precision.HIGHEST)
  s_ref[...] = n2 - 2.0 * g

  iota = lax.broadcasted_iota(jnp.int32, (8, N2), 1)

  def blk(j, carry):
    sb = s_ref[pl.ds(j * 8, 8), :]
    cols = []
    for r in range(KNN):
      m = jnp.min(sb, axis=1, keepdims=True)
      am = jnp.min(jnp.where(sb == m, iota, N2), axis=1,
                   keepdims=True)
      cols.append(am)
      if r < KNN - 1:
        sb = jnp.where(iota == am, jnp.inf, sb)
    idxb = jnp.concatenate(cols, axis=1)
    idx_ref[0, pl.ds(j * 8, 8), :] = idxb + b * N2
    return carry

  lax.fori_loop(0, TI // 8, blk, 0)

  f1 = feat1_ref[0]
  bt = (lax.dot_general(f1, w0[:, :C], (((0,), (1,)), ((), ())),
                        preferred_element_type=jnp.float32,
                      precision=lax.Precision.HIGHEST)
        - lax.dot_general(x1, w0[:, 2 * C:], (((0,), (1,)), ((), ())),
                          preferred_element_type=jnp.float32,
                      precision=lax.Precision.HIGHEST)
        + b0_ref[...])
  base_ref[0] = bt

  @pl.when(it == 0)
  def _():
    f2 = feat2_ref[0]
    pcv = (lax.dot_general(f2, w0[:, C:2 * C], (((0,), (1,)), ((), ())),
                           preferred_element_type=jnp.float32,
                      precision=lax.Precision.HIGHEST)
           + lax.dot_general(x2, w0[:, 2 * C:], (((0,), (1,)), ((), ())),
                             preferred_element_type=jnp.float32,
                      precision=lax.Precision.HIGHEST))
    pc_ref[0] = pcv


def _knn_proj_call(xyz1, xyz2, feat1, feat2, w0, b0_2d):
  return pl.pallas_call(
      _knn_proj_body,
      grid=(B, N // TI),
      in_specs=[
          pl.BlockSpec((1, 3, TI), lambda b, it: (b, 0, it)),
          pl.BlockSpec((1, 3, N2), lambda b, it: (b, 0, 0)),
          pl.BlockSpec((1, C, TI), lambda b, it: (b, 0, it)),
          pl.BlockSpec((1, C, N2), lambda b, it: (b, 0, 0)),
          pl.BlockSpec((C, 131), lambda b, it: (0, 0)),
          pl.BlockSpec((1, C), lambda b, it: (0, 0)),
      ],
      out_specs=[
          pl.BlockSpec((1, TI, KNN), lambda b, it: (b, it, 0)),
          pl.BlockSpec((1, TI, C), lambda b, it: (b, it, 0)),
          pl.BlockSpec((1, N2, C), lambda b, it: (b, 0, 0)),
      ],
      out_shape=[
          jax.ShapeDtypeStruct((B, N, KNN), jnp.int32),
          jax.ShapeDtypeStruct((B, N, C), jnp.float32),
          jax.ShapeDtypeStruct((B, N2, C), jnp.float32),
      ],
      scratch_shapes=[pltpu.VMEM((TI, N2), jnp.float32)],
      compiler_params=pltpu.CompilerParams(
          dimension_semantics=("arbitrary", "arbitrary")),
  )(xyz1, xyz2, feat1, feat2, w0, b0_2d)


def _sc_gather_body(table_hbm, idx_hbm, out_hbm, idx_v, rows_v, sem):
  wid = lax.axis_index("s") * 2 + lax.axis_index("c")
  base = wid * ROWS_PER_W

  def chunk(ci, carry):
    off = base + ci * SC_CHUNK
    pltpu.sync_copy(idx_hbm.at[pl.ds(off, SC_CHUNK)], idx_v)
    pltpu.async_copy(table_hbm.at[idx_v], rows_v, sem).wait()
    pltpu.sync_copy(rows_v, out_hbm.at[pl.ds(off, SC_CHUNK)])
    return carry

  lax.fori_loop(0, SC_NCHUNK, chunk, 0)


@functools.cache
def _get_sc_gather():
  return pl.kernel(
      _sc_gather_body,
      out_type=jax.ShapeDtypeStruct((TOTAL_ROWS, C), jnp.float32),
      mesh=plsc.VectorSubcoreMesh(core_axis_name="c", subcore_axis_name="s"),
      scratch_types=[
          pltpu.VMEM((SC_CHUNK,), jnp.int32),
          pltpu.VMEM((SC_CHUNK, C), jnp.float32),
          pltpu.SemaphoreType.DMA,
      ],
      compiler_params=pltpu.CompilerParams(use_tc_tiling_on_sc=False),
  )


def _group_mat():
  i = lax.broadcasted_iota(jnp.int32, (C, C), 0)
  j = lax.broadcasted_iota(jnp.int32, (C, C), 1)
  return ((i // 16) == (j // 16)).astype(jnp.float32)


def _mlp_body(g_ref, base_ref, w1_ref, b1_ref, g0_ref, beta0_ref,
              g1_ref, beta1_ref, out_ref, z_ref):
  nt = (N * KNN) // TT
  qt = TT // KNN
  gm = _group_mat()
  cnt = 16.0 * N * KNN

  def tile_y(t):
    gt = g_ref[0, pl.ds(t * TT, TT), :]
    bt = base_ref[0, pl.ds(t * qt, qt), :]
    y = gt.reshape(qt, KNN, C) + bt[:, None, :]
    return y.reshape(TT, C)

  def p1(t, carry):
    s, q = carry
    y = tile_y(t)
    return (s + jnp.sum(y, axis=0, keepdims=True),
            q + jnp.sum(y * y, axis=0, keepdims=True))

  z1 = jnp.zeros((1, C), jnp.float32)
  s0, q0 = lax.fori_loop(0, nt, p1, (z1, z1))
  mean0 = jnp.dot(s0, gm, preferred_element_type=jnp.float32,
                      precision=lax.Precision.HIGHEST) / cnt
  var0 = jnp.dot(q0, gm, preferred_element_type=jnp.float32,
                      precision=lax.Precision.HIGHEST) / cnt - mean0 * mean0
  inv0 = lax.rsqrt(var0 + EPS)
  sc0 = inv0 * g0_ref[...]
  sh0 = beta0_ref[...] - mean0 * sc0

  w1 = w1_ref[...]
  b1 = b1_ref[...]

  def p2(t, carry):
    s, q = carry
    ya = tile_y(t) * sc0 + sh0
    ya = jnp.where(ya >= 0, ya, NEG_SLOPE * ya)
    z = lax.dot_general(ya, w1, (((1,), (1,)), ((), ())),
                        preferred_element_type=jnp.float32,
                      precision=lax.Precision.HIGHEST) + b1
    z_ref[pl.ds(t * TT, TT), :] = z
    return (s + jnp.sum(z, axis=0, keepdims=True),
            q + jnp.sum(z * z, axis=0, keepdims=True))

  s1, q1 = lax.fori_loop(0, nt, p2, (z1, z1))
  mean1 = jnp.dot(s1, gm, preferred_element_type=jnp.float32,
                      precision=lax.Precision.HIGHEST) / cnt
  var1 = jnp.dot(q1, gm, preferred_element_type=jnp.float32,
                      precision=lax.Precision.HIGHEST) / cnt - mean1 * mean1
  inv1 = lax.rsqrt(var1 + EPS)
  sc1 = inv1 * g1_ref[...]
  sh1 = beta1_ref[...] - mean1 * sc1

  def p3(t, carry):
    z = z_ref[pl.ds(t * TT, TT), :]
    za = z * sc1 + sh1
    za = jnp.where(za >= 0, za, NEG_SLOPE * za)
    zm = jnp.max(za.reshape(qt, KNN, C), axis=1)
    out_ref[0, :, pl.ds(t * qt, qt)] = zm.T
    return carry

  lax.fori_loop(0, nt, p3, 0)


def _mlp_call(g, base, w1, b1_2d, g0_2d, beta0_2d, g1_2d, beta1_2d):
  return pl.pallas_call(
      _mlp_body,
      grid=(B,),
      in_specs=[
          pl.BlockSpec((1, N * KNN, C), lambda b: (b, 0, 0)),
          pl.BlockSpec((1, N, C), lambda b: (b, 0, 0)),
          pl.BlockSpec((C, C), lambda b: (0, 0)),
          pl.BlockSpec((1, C), lambda b: (0, 0)),
          pl.BlockSpec((1, C), lambda b: (0, 0)),
          pl.BlockSpec((1, C), lambda b: (0, 0)),
          pl.BlockSpec((1, C), lambda b: (0, 0)),
          pl.BlockSpec((1, C), lambda b: (0, 0)),
      ],
      out_specs=pl.BlockSpec((1, C, N), lambda b: (b, 0, 0)),
      out_shape=jax.ShapeDtypeStruct((B, C, N), jnp.float32),
      scratch_shapes=[pltpu.VMEM((N * KNN, C), jnp.float32)],
      compiler_params=pltpu.CompilerParams(
          dimension_semantics=("arbitrary",)),
  )(g, base, w1, b1_2d, g0_2d, beta0_2d, g1_2d, beta1_2d)


def kernel(xyz1, xyz2, feat1, feat2, W0, b0, g0, beta0, W1, b1, g1, beta1):
  idx, base, pc = _knn_proj_call(xyz1, xyz2, feat1, feat2, W0,
                                 b0.reshape(1, C))
  gathered = _get_sc_gather()(pc.reshape(B * N2, C), idx.reshape(TOTAL_ROWS))
  return _mlp_call(gathered.reshape(B, N * KNN, C), base, W1,
                   b1.reshape(1, C), g0.reshape(1, C), beta0.reshape(1, C),
                   g1.reshape(1, C), beta1.reshape(1, C))


# speedup vs baseline: 4.2097x; 2.6224x over previous
import functools

import jax
import jax.numpy as jnp
from jax import lax
from jax.experimental import pallas as pl
from jax.experimental.pallas import tpu as pltpu
from jax.experimental.pallas import tpu_sc as plsc

KNN = 16
B, N, N2 = 8, 2048, 2048
C = 64
TI = 256
TT = 4096
EPS = 1e-5
NEG_SLOPE = 0.1

SC_WORKERS = 32
TOTAL_ROWS = B * N * KNN
ROWS_PER_W = TOTAL_ROWS // SC_WORKERS
SC_CHUNK = 128
SC_NCHUNK = ROWS_PER_W // SC_CHUNK


def _knn_proj_body(xyz1_ref, xyz2_ref, feat1_ref, feat2_ref, w0_ref, b0_ref,
                   idx_ref, base_ref, pc_ref, s_ref):
  b = pl.program_id(0)
  it = pl.program_id(1)
  x1 = xyz1_ref[0]
  x2 = xyz2_ref[0]
  w0 = w0_ref[...]

  n2 = jnp.sum(x2 * x2, axis=0, keepdims=True)
  g = lax.dot_general(x1, x2, (((0,), (0,)), ((), ())),
                      preferred_element_type=jnp.float32,
                      precision=lax.Precision.HIGHEST)
  s_ref[...] = n2 - 2.0 * g

  lane = lax.broadcasted_iota(jnp.int32, (8, 1, 128), 2)
  nvr = N2 // 128

  def topk8(sb3):
    cols = []
    for r in range(KNN):
      m = jnp.min(sb3, axis=1)
      gv = jnp.min(m, axis=1, keepdims=True)[:, :, None]
      em = sb3 == gv
      cv = jnp.min(jnp.where(
          em, lax.broadcasted_iota(jnp.int32, (8, nvr, 128), 1),
          nvr), axis=1, keepdims=True)
      gj = jnp.min(jnp.where(cv < nvr, cv * 128 + lane, N2),
                   axis=2, keepdims=False)
      cols.append(gj)
      if r < KNN - 1:
        sb3 = jnp.where(em, jnp.inf, sb3)
    return jnp.concatenate(cols, axis=1)

  def blk(j, carry):
    for u in range(2):
      sb3 = s_ref[pl.ds(j * 16 + u * 8, 8), :].reshape(8, nvr, 128)
      idx_ref[0, pl.ds(j * 16 + u * 8, 8), :] = topk8(sb3) + b * N2
    return carry

  lax.fori_loop(0, TI // 16, blk, 0)

  f1 = feat1_ref[0]
  bt = (lax.dot_general(f1, w0[:, :C], (((0,), (1,)), ((), ())),
                        preferred_element_type=jnp.float32,
                      precision=lax.Precision.HIGHEST)
        - lax.dot_general(x1, w0[:, 2 * C:], (((0,), (1,)), ((), ())),
                          preferred_element_type=jnp.float32,
                      precision=lax.Precision.HIGHEST)
        + b0_ref[...])
  base_ref[0] = bt

  @pl.when(it == 0)
  def _():
    f2 = feat2_ref[0]
    pcv = (lax.dot_general(f2, w0[:, C:2 * C], (((0,), (1,)), ((), ())),
                           preferred_element_type=jnp.float32,
                      precision=lax.Precision.HIGHEST)
           + lax.dot_general(x2, w0[:, 2 * C:], (((0,), (1,)), ((), ())),
                             preferred_element_type=jnp.float32,
                      precision=lax.Precision.HIGHEST))
    pc_ref[0] = pcv


def _knn_proj_call(xyz1, xyz2, feat1, feat2, w0, b0_2d):
  return pl.pallas_call(
      _knn_proj_body,
      grid=(B, N // TI),
      in_specs=[
          pl.BlockSpec((1, 3, TI), lambda b, it: (b, 0, it)),
          pl.BlockSpec((1, 3, N2), lambda b, it: (b, 0, 0)),
          pl.BlockSpec((1, C, TI), lambda b, it: (b, 0, it)),
          pl.BlockSpec((1, C, N2), lambda b, it: (b, 0, 0)),
          pl.BlockSpec((C, 131), lambda b, it: (0, 0)),
          pl.BlockSpec((1, C), lambda b, it: (0, 0)),
      ],
      out_specs=[
          pl.BlockSpec((1, TI, KNN), lambda b, it: (b, it, 0)),
          pl.BlockSpec((1, TI, C), lambda b, it: (b, it, 0)),
          pl.BlockSpec((1, N2, C), lambda b, it: (b, 0, 0)),
      ],
      out_shape=[
          jax.ShapeDtypeStruct((B, N, KNN), jnp.int32),
          jax.ShapeDtypeStruct((B, N, C), jnp.float32),
          jax.ShapeDtypeStruct((B, N2, C), jnp.float32),
      ],
      scratch_shapes=[pltpu.VMEM((TI, N2), jnp.float32)],
      compiler_params=pltpu.CompilerParams(
          dimension_semantics=("arbitrary", "arbitrary")),
  )(xyz1, xyz2, feat1, feat2, w0, b0_2d)


def _sc_gather_body(table_hbm, idx_hbm, out_hbm, idx_v, rows_v, sem):
  wid = lax.axis_index("s") * 2 + lax.axis_index("c")
  base = wid * ROWS_PER_W

  def chunk(ci, carry):
    off = base + ci * SC_CHUNK
    pltpu.sync_copy(idx_hbm.at[pl.ds(off, SC_CHUNK)], idx_v)
    pltpu.async_copy(table_hbm.at[idx_v], rows_v, sem).wait()
    pltpu.sync_copy(rows_v, out_hbm.at[pl.ds(off, SC_CHUNK)])
    return carry

  lax.fori_loop(0, SC_NCHUNK, chunk, 0)


@functools.cache
def _get_sc_gather():
  return pl.kernel(
      _sc_gather_body,
      out_type=jax.ShapeDtypeStruct((TOTAL_ROWS, C), jnp.float32),
      mesh=plsc.VectorSubcoreMesh(core_axis_name="c", subcore_axis_name="s"),
      scratch_types=[
          pltpu.VMEM((SC_CHUNK,), jnp.int32),
          pltpu.VMEM((SC_CHUNK, C), jnp.float32),
          pltpu.SemaphoreType.DMA,
      ],
      compiler_params=pltpu.CompilerParams(use_tc_tiling_on_sc=False),
  )


def _group_mat():
  i = lax.broadcasted_iota(jnp.int32, (C, C), 0)
  j = lax.broadcasted_iota(jnp.int32, (C, C), 1)
  return ((i // 16) == (j // 16)).astype(jnp.float32)


def _mlp_body(g_ref, base_ref, w1_ref, b1_ref, g0_ref, beta0_ref,
              g1_ref, beta1_ref, out_ref, z_ref):
  nt = (N * KNN) // TT
  qt = TT // KNN
  gm = _group_mat()
  cnt = 16.0 * N * KNN

  def tile_y(t):
    gt = g_ref[0, pl.ds(t * TT, TT), :]
    bt = base_ref[0, pl.ds(t * qt, qt), :]
    y = gt.reshape(qt, KNN, C) + bt[:, None, :]
    return y.reshape(TT, C)

  def p1(t, carry):
    s, q = carry
    y = tile_y(t)
    return (s + jnp.sum(y, axis=0, keepdims=True),
            q + jnp.sum(y * y, axis=0, keepdims=True))

  z1 = jnp.zeros((1, C), jnp.float32)
  s0, q0 = lax.fori_loop(0, nt, p1, (z1, z1))
  mean0 = jnp.dot(s0, gm, preferred_element_type=jnp.float32,
                      precision=lax.Precision.HIGHEST) / cnt
  var0 = jnp.dot(q0, gm, preferred_element_type=jnp.float32,
                      precision=lax.Precision.HIGHEST) / cnt - mean0 * mean0
  inv0 = lax.rsqrt(var0 + EPS)
  sc0 = inv0 * g0_ref[...]
  sh0 = beta0_ref[...] - mean0 * sc0

  w1 = w1_ref[...]
  b1 = b1_ref[...]

  def p2(t, carry):
    s, q = carry
    ya = tile_y(t) * sc0 + sh0
    ya = jnp.where(ya >= 0, ya, NEG_SLOPE * ya)
    z = lax.dot_general(ya, w1, (((1,), (1,)), ((), ())),
                        preferred_element_type=jnp.float32,
                      precision=lax.Precision.HIGHEST) + b1
    z_ref[pl.ds(t * TT, TT), :] = z
    return (s + jnp.sum(z, axis=0, keepdims=True),
            q + jnp.sum(z * z, axis=0, keepdims=True))

  s1, q1 = lax.fori_loop(0, nt, p2, (z1, z1))
  mean1 = jnp.dot(s1, gm, preferred_element_type=jnp.float32,
                      precision=lax.Precision.HIGHEST) / cnt
  var1 = jnp.dot(q1, gm, preferred_element_type=jnp.float32,
                      precision=lax.Precision.HIGHEST) / cnt - mean1 * mean1
  inv1 = lax.rsqrt(var1 + EPS)
  sc1 = inv1 * g1_ref[...]
  sh1 = beta1_ref[...] - mean1 * sc1

  def p3(t, carry):
    z = z_ref[pl.ds(t * TT, TT), :]
    za = z * sc1 + sh1
    za = jnp.where(za >= 0, za, NEG_SLOPE * za)
    zm = jnp.max(za.reshape(qt, KNN, C), axis=1)
    out_ref[0, :, pl.ds(t * qt, qt)] = zm.T
    return carry

  lax.fori_loop(0, nt, p3, 0)


def _mlp_call(g, base, w1, b1_2d, g0_2d, beta0_2d, g1_2d, beta1_2d):
  return pl.pallas_call(
      _mlp_body,
      grid=(B,),
      in_specs=[
          pl.BlockSpec((1, N * KNN, C), lambda b: (b, 0, 0)),
          pl.BlockSpec((1, N, C), lambda b: (b, 0, 0)),
          pl.BlockSpec((C, C), lambda b: (0, 0)),
          pl.BlockSpec((1, C), lambda b: (0, 0)),
          pl.BlockSpec((1, C), lambda b: (0, 0)),
          pl.BlockSpec((1, C), lambda b: (0, 0)),
          pl.BlockSpec((1, C), lambda b: (0, 0)),
          pl.BlockSpec((1, C), lambda b: (0, 0)),
      ],
      out_specs=pl.BlockSpec((1, C, N), lambda b: (b, 0, 0)),
      out_shape=jax.ShapeDtypeStruct((B, C, N), jnp.float32),
      scratch_shapes=[pltpu.VMEM((N * KNN, C), jnp.float32)],
      compiler_params=pltpu.CompilerParams(
          dimension_semantics=("arbitrary",)),
  )(g, base, w1, b1_2d, g0_2d, beta0_2d, g1_2d, beta1_2d)


def kernel(xyz1, xyz2, feat1, feat2, W0, b0, g0, beta0, W1, b1, g1, beta1):
  idx, base, pc = _knn_proj_call(xyz1, xyz2, feat1, feat2, W0,
                                 b0.reshape(1, C))
  gathered = _get_sc_gather()(pc.reshape(B * N2, C), idx.reshape(TOTAL_ROWS))
  return _mlp_call(gathered.reshape(B, N * KNN, C), base, W1,
                   b1.reshape(1, C), g0.reshape(1, C), beta0.reshape(1, C),
                   g1.reshape(1, C), beta1.reshape(1, C))


# speedup vs baseline: 5.9729x; 1.4188x over previous
import functools

import jax
import jax.numpy as jnp
from jax import lax
from jax.experimental import pallas as pl
from jax.experimental.pallas import tpu as pltpu
from jax.experimental.pallas import tpu_sc as plsc

KNN = 16
B, N, N2 = 8, 2048, 2048
C = 64
TI = 256
TT = 4096
EPS = 1e-5
NEG_SLOPE = 0.1

SC_WORKERS = 32
TOTAL_ROWS = B * N * KNN
ROWS_PER_W = TOTAL_ROWS // SC_WORKERS
SC_CHUNK = 128
SC_NCHUNK = ROWS_PER_W // SC_CHUNK


def _knn_proj_body(xyz1_ref, xyz2_ref, feat1_ref, feat2_ref, w0_ref, b0_ref,
                   idx_ref, base_ref, pc_ref, s_ref):
  b = pl.program_id(0)
  it = pl.program_id(1)
  x1 = xyz1_ref[0]
  x2 = xyz2_ref[0]
  w0 = w0_ref[...]

  n2 = jnp.sum(x2 * x2, axis=0, keepdims=True)
  g = lax.dot_general(x1, x2, (((0,), (0,)), ((), ())),
                      preferred_element_type=jnp.float32,
                      precision=lax.Precision.HIGHEST)
  s_ref[...] = n2 - 2.0 * g

  lane = lax.broadcasted_iota(jnp.int32, (8, 128), 1)
  nvr = N2 // 128
  inf = jnp.float32(jnp.inf)

  def topk8(sb):
    m = functools.reduce(jnp.minimum, sb)
    cols = []
    for r in range(KNN):
      gv = jnp.min(m, axis=1, keepdims=True)
      cv = jnp.full((8, 128), nvr, jnp.int32)
      mn = jnp.full((8, 128), inf, jnp.float32)
      for v in range(nvr):
        emv = sb[v] == gv
        cv = jnp.minimum(cv, jnp.where(emv, v, nvr))
        if r < KNN - 1:
          sb[v] = jnp.where(emv, inf, sb[v])
          mn = jnp.minimum(mn, sb[v])
      m = mn
      gj = jnp.min(jnp.where(cv < nvr, cv * 128 + lane, N2),
                   axis=1, keepdims=True)
      cols.append(gj)
    return jnp.concatenate(cols, axis=1)

  def blk(j, carry):
    for u in range(2):
      row = j * 16 + u * 8
      sbw = s_ref[pl.ds(row, 8), :]
      sb = [sbw[:, v * 128:(v + 1) * 128] for v in range(nvr)]
      idx_ref[0, pl.ds(row, 8), :] = topk8(sb) + b * N2
    return carry

  lax.fori_loop(0, TI // 16, blk, 0)

  f1 = feat1_ref[0]
  bt = (lax.dot_general(f1, w0[:, :C], (((0,), (1,)), ((), ())),
                        preferred_element_type=jnp.float32,
                      precision=lax.Precision.HIGHEST)
        - lax.dot_general(x1, w0[:, 2 * C:], (((0,), (1,)), ((), ())),
                          preferred_element_type=jnp.float32,
                      precision=lax.Precision.HIGHEST)
        + b0_ref[...])
  base_ref[0] = bt

  @pl.when(it == 0)
  def _():
    f2 = feat2_ref[0]
    pcv = (lax.dot_general(f2, w0[:, C:2 * C], (((0,), (1,)), ((), ())),
                           preferred_element_type=jnp.float32,
                      precision=lax.Precision.HIGHEST)
           + lax.dot_general(x2, w0[:, 2 * C:], (((0,), (1,)), ((), ())),
                             preferred_element_type=jnp.float32,
                      precision=lax.Precision.HIGHEST))
    pc_ref[0] = pcv


def _knn_proj_call(xyz1, xyz2, feat1, feat2, w0, b0_2d):
  return pl.pallas_call(
      _knn_proj_body,
      grid=(B, N // TI),
      in_specs=[
          pl.BlockSpec((1, 3, TI), lambda b, it: (b, 0, it)),
          pl.BlockSpec((1, 3, N2), lambda b, it: (b, 0, 0)),
          pl.BlockSpec((1, C, TI), lambda b, it: (b, 0, it)),
          pl.BlockSpec((1, C, N2), lambda b, it: (b, 0, 0)),
          pl.BlockSpec((C, 131), lambda b, it: (0, 0)),
          pl.BlockSpec((1, C), lambda b, it: (0, 0)),
      ],
      out_specs=[
          pl.BlockSpec((1, TI, KNN), lambda b, it: (b, it, 0)),
          pl.BlockSpec((1, TI, C), lambda b, it: (b, it, 0)),
          pl.BlockSpec((1, N2, C), lambda b, it: (b, 0, 0)),
      ],
      out_shape=[
          jax.ShapeDtypeStruct((B, N, KNN), jnp.int32),
          jax.ShapeDtypeStruct((B, N, C), jnp.float32),
          jax.ShapeDtypeStruct((B, N2, C), jnp.float32),
      ],
      scratch_shapes=[pltpu.VMEM((TI, N2), jnp.float32)],
      compiler_params=pltpu.CompilerParams(
          dimension_semantics=("arbitrary", "arbitrary")),
  )(xyz1, xyz2, feat1, feat2, w0, b0_2d)


def _sc_gather_body(table_hbm, idx_hbm, out_hbm, idx_v, rows_v, sem):
  wid = lax.axis_index("s") * 2 + lax.axis_index("c")
  base = wid * ROWS_PER_W

  def chunk(ci, carry):
    off = base + ci * SC_CHUNK
    pltpu.sync_copy(idx_hbm.at[pl.ds(off, SC_CHUNK)], idx_v)
    pltpu.async_copy(table_hbm.at[idx_v], rows_v, sem).wait()
    pltpu.sync_copy(rows_v, out_hbm.at[pl.ds(off, SC_CHUNK)])
    return carry

  lax.fori_loop(0, SC_NCHUNK, chunk, 0)


@functools.cache
def _get_sc_gather():
  return pl.kernel(
      _sc_gather_body,
      out_type=jax.ShapeDtypeStruct((TOTAL_ROWS, C), jnp.float32),
      mesh=plsc.VectorSubcoreMesh(core_axis_name="c", subcore_axis_name="s"),
      scratch_types=[
          pltpu.VMEM((SC_CHUNK,), jnp.int32),
          pltpu.VMEM((SC_CHUNK, C), jnp.float32),
          pltpu.SemaphoreType.DMA,
      ],
      compiler_params=pltpu.CompilerParams(use_tc_tiling_on_sc=False),
  )


def _group_mat():
  i = lax.broadcasted_iota(jnp.int32, (C, C), 0)
  j = lax.broadcasted_iota(jnp.int32, (C, C), 1)
  return ((i // 16) == (j // 16)).astype(jnp.float32)


def _mlp_body(g_ref, base_ref, w1_ref, b1_ref, g0_ref, beta0_ref,
              g1_ref, beta1_ref, out_ref, z_ref):
  nt = (N * KNN) // TT
  qt = TT // KNN
  gm = _group_mat()
  cnt = 16.0 * N * KNN

  def tile_y(t):
    gt = g_ref[0, pl.ds(t * TT, TT), :]
    bt = base_ref[0, pl.ds(t * qt, qt), :]
    y = gt.reshape(qt, KNN, C) + bt[:, None, :]
    return y.reshape(TT, C)

  def p1(t, carry):
    s, q = carry
    y = tile_y(t)
    return (s + jnp.sum(y, axis=0, keepdims=True),
            q + jnp.sum(y * y, axis=0, keepdims=True))

  z1 = jnp.zeros((1, C), jnp.float32)
  s0, q0 = lax.fori_loop(0, nt, p1, (z1, z1))
  mean0 = jnp.dot(s0, gm, preferred_element_type=jnp.float32,
                      precision=lax.Precision.HIGHEST) / cnt
  var0 = jnp.dot(q0, gm, preferred_element_type=jnp.float32,
                      precision=lax.Precision.HIGHEST) / cnt - mean0 * mean0
  inv0 = lax.rsqrt(var0 + EPS)
  sc0 = inv0 * g0_ref[...]
  sh0 = beta0_ref[...] - mean0 * sc0

  w1 = w1_ref[...]
  b1 = b1_ref[...]

  def p2(t, carry):
    s, q = carry
    ya = tile_y(t) * sc0 + sh0
    ya = jnp.where(ya >= 0, ya, NEG_SLOPE * ya)
    z = lax.dot_general(ya, w1, (((1,), (1,)), ((), ())),
                        preferred_element_type=jnp.float32,
                      precision=lax.Precision.HIGHEST) + b1
    z_ref[pl.ds(t * TT, TT), :] = z
    return (s + jnp.sum(z, axis=0, keepdims=True),
            q + jnp.sum(z * z, axis=0, keepdims=True))

  s1, q1 = lax.fori_loop(0, nt, p2, (z1, z1))
  mean1 = jnp.dot(s1, gm, preferred_element_type=jnp.float32,
                      precision=lax.Precision.HIGHEST) / cnt
  var1 = jnp.dot(q1, gm, preferred_element_type=jnp.float32,
                      precision=lax.Precision.HIGHEST) / cnt - mean1 * mean1
  inv1 = lax.rsqrt(var1 + EPS)
  sc1 = inv1 * g1_ref[...]
  sh1 = beta1_ref[...] - mean1 * sc1

  def p3(t, carry):
    z = z_ref[pl.ds(t * TT, TT), :]
    za = z * sc1 + sh1
    za = jnp.where(za >= 0, za, NEG_SLOPE * za)
    zm = jnp.max(za.reshape(qt, KNN, C), axis=1)
    out_ref[0, :, pl.ds(t * qt, qt)] = zm.T
    return carry

  lax.fori_loop(0, nt, p3, 0)


def _mlp_call(g, base, w1, b1_2d, g0_2d, beta0_2d, g1_2d, beta1_2d):
  return pl.pallas_call(
      _mlp_body,
      grid=(B,),
      in_specs=[
          pl.BlockSpec((1, N * KNN, C), lambda b: (b, 0, 0)),
          pl.BlockSpec((1, N, C), lambda b: (b, 0, 0)),
          pl.BlockSpec((C, C), lambda b: (0, 0)),
          pl.BlockSpec((1, C), lambda b: (0, 0)),
          pl.BlockSpec((1, C), lambda b: (0, 0)),
          pl.BlockSpec((1, C), lambda b: (0, 0)),
          pl.BlockSpec((1, C), lambda b: (0, 0)),
          pl.BlockSpec((1, C), lambda b: (0, 0)),
      ],
      out_specs=pl.BlockSpec((1, C, N), lambda b: (b, 0, 0)),
      out_shape=jax.ShapeDtypeStruct((B, C, N), jnp.float32),
      scratch_shapes=[pltpu.VMEM((N * KNN, C), jnp.float32)],
      compiler_params=pltpu.CompilerParams(
          dimension_semantics=("arbitrary",)),
  )(g, base, w1, b1_2d, g0_2d, beta0_2d, g1_2d, beta1_2d)


def kernel(xyz1, xyz2, feat1, feat2, W0, b0, g0, beta0, W1, b1, g1, beta1):
  idx, base, pc = _knn_proj_call(xyz1, xyz2, feat1, feat2, W0,
                                 b0.reshape(1, C))
  gathered = _get_sc_gather()(pc.reshape(B * N2, C), idx.reshape(TOTAL_ROWS))
  return _mlp_call(gathered.reshape(B, N * KNN, C), base, W1,
                   b1.reshape(1, C), g0.reshape(1, C), beta0.reshape(1, C),
                   g1.reshape(1, C), beta1.reshape(1, C))


# speedup vs baseline: 6.1909x; 1.0365x over previous
import functools

import jax
import jax.numpy as jnp
from jax import lax
from jax.experimental import pallas as pl
from jax.experimental.pallas import tpu as pltpu
from jax.experimental.pallas import tpu_sc as plsc

KNN = 16
B, N, N2 = 8, 2048, 2048
C = 64
TI = 256
TT = 4096
EPS = 1e-5
NEG_SLOPE = 0.1

SC_WORKERS = 32
TOTAL_ROWS = B * N * KNN
ROWS_PER_W = TOTAL_ROWS // SC_WORKERS
SC_CHUNK = 128
SC_NCHUNK = ROWS_PER_W // SC_CHUNK


def _knn_proj_body(xyz1_ref, xyz2_ref, feat1_ref, feat2_ref, w0_ref, b0_ref,
                   idx_ref, base_ref, pc_ref, s_ref):
  b = pl.program_id(0)
  it = pl.program_id(1)
  x1 = xyz1_ref[0]
  x2 = xyz2_ref[0]
  w0 = w0_ref[...]

  n2 = jnp.sum(x2 * x2, axis=0, keepdims=True)
  g = lax.dot_general(x1, x2, (((0,), (0,)), ((), ())),
                      preferred_element_type=jnp.float32,
                      precision=lax.Precision.HIGHEST)
  s_ref[...] = n2 - 2.0 * g

  lane = lax.broadcasted_iota(jnp.int32, (8, 128), 1)
  nvr = N2 // 128
  inf = jnp.float32(jnp.inf)

  def topk8(sb):
    m = functools.reduce(jnp.minimum, sb)
    cols = []
    for r in range(KNN):
      gv = jnp.min(m, axis=1, keepdims=True)
      mparts, jparts = [], []
      for g0 in range(0, nvr, 4):
        mp, jp = [], []
        for v in range(g0, g0 + 4):
          emv = sb[v] == gv
          jp.append(jnp.where(emv, lane + v * 128, N2))
          if r < KNN - 1:
            sb[v] = jnp.where(emv, inf, sb[v])
            mp.append(sb[v])
        jparts.append(jnp.minimum(jnp.minimum(jp[0], jp[1]),
                                  jnp.minimum(jp[2], jp[3])))
        if r < KNN - 1:
          mparts.append(jnp.minimum(jnp.minimum(mp[0], mp[1]),
                                    jnp.minimum(mp[2], mp[3])))
      if r < KNN - 1:
        m = jnp.minimum(jnp.minimum(mparts[0], mparts[1]),
                        jnp.minimum(mparts[2], mparts[3]))
      gj = functools.reduce(jnp.minimum, jparts)
      cols.append(jnp.min(gj, axis=1, keepdims=True))
    return jnp.concatenate(cols, axis=1)

  def blk(j, carry):
    for u in range(2):
      row = j * 16 + u * 8
      sbw = s_ref[pl.ds(row, 8), :]
      sb = [sbw[:, v * 128:(v + 1) * 128] for v in range(nvr)]
      idx_ref[0, pl.ds(row, 8), :] = topk8(sb) + b * N2
    return carry

  lax.fori_loop(0, TI // 16, blk, 0)

  f1 = feat1_ref[0]
  bt = (lax.dot_general(f1, w0[:, :C], (((0,), (1,)), ((), ())),
                        preferred_element_type=jnp.float32,
                      precision=lax.Precision.HIGHEST)
        - lax.dot_general(x1, w0[:, 2 * C:], (((0,), (1,)), ((), ())),
                          preferred_element_type=jnp.float32,
                      precision=lax.Precision.HIGHEST)
        + b0_ref[...])
  base_ref[0] = bt

  @pl.when(it == 0)
  def _():
    f2 = feat2_ref[0]
    pcv = (lax.dot_general(f2, w0[:, C:2 * C], (((0,), (1,)), ((), ())),
                           preferred_element_type=jnp.float32,
                      precision=lax.Precision.HIGHEST)
           + lax.dot_general(x2, w0[:, 2 * C:], (((0,), (1,)), ((), ())),
                             preferred_element_type=jnp.float32,
                      precision=lax.Precision.HIGHEST))
    pc_ref[0] = pcv


def _knn_proj_call(xyz1, xyz2, feat1, feat2, w0, b0_2d):
  return pl.pallas_call(
      _knn_proj_body,
      grid=(B, N // TI),
      in_specs=[
          pl.BlockSpec((1, 3, TI), lambda b, it: (b, 0, it)),
          pl.BlockSpec((1, 3, N2), lambda b, it: (b, 0, 0)),
          pl.BlockSpec((1, C, TI), lambda b, it: (b, 0, it)),
          pl.BlockSpec((1, C, N2), lambda b, it: (b, 0, 0)),
          pl.BlockSpec((C, 131), lambda b, it: (0, 0)),
          pl.BlockSpec((1, C), lambda b, it: (0, 0)),
      ],
      out_specs=[
          pl.BlockSpec((1, TI, KNN), lambda b, it: (b, it, 0)),
          pl.BlockSpec((1, TI, C), lambda b, it: (b, it, 0)),
          pl.BlockSpec((1, N2, C), lambda b, it: (b, 0, 0)),
      ],
      out_shape=[
          jax.ShapeDtypeStruct((B, N, KNN), jnp.int32),
          jax.ShapeDtypeStruct((B, N, C), jnp.float32),
          jax.ShapeDtypeStruct((B, N2, C), jnp.float32),
      ],
      scratch_shapes=[pltpu.VMEM((TI, N2), jnp.float32)],
      compiler_params=pltpu.CompilerParams(
          dimension_semantics=("arbitrary", "arbitrary")),
  )(xyz1, xyz2, feat1, feat2, w0, b0_2d)


def _sc_gather_body(table_hbm, idx_hbm, out_hbm, idx_v, rows_v, sem):
  wid = lax.axis_index("s") * 2 + lax.axis_index("c")
  base = wid * ROWS_PER_W

  def chunk(ci, carry):
    off = base + ci * SC_CHUNK
    pltpu.sync_copy(idx_hbm.at[pl.ds(off, SC_CHUNK)], idx_v)
    pltpu.async_copy(table_hbm.at[idx_v], rows_v, sem).wait()
    pltpu.sync_copy(rows_v, out_hbm.at[pl.ds(off, SC_CHUNK)])
    return carry

  lax.fori_loop(0, SC_NCHUNK, chunk, 0)


@functools.cache
def _get_sc_gather():
  return pl.kernel(
      _sc_gather_body,
      out_type=jax.ShapeDtypeStruct((TOTAL_ROWS, C), jnp.float32),
      mesh=plsc.VectorSubcoreMesh(core_axis_name="c", subcore_axis_name="s"),
      scratch_types=[
          pltpu.VMEM((SC_CHUNK,), jnp.int32),
          pltpu.VMEM((SC_CHUNK, C), jnp.float32),
          pltpu.SemaphoreType.DMA,
      ],
      compiler_params=pltpu.CompilerParams(use_tc_tiling_on_sc=False),
  )


def _group_mat():
  i = lax.broadcasted_iota(jnp.int32, (C, C), 0)
  j = lax.broadcasted_iota(jnp.int32, (C, C), 1)
  return ((i // 16) == (j // 16)).astype(jnp.float32)


def _mlp_body(g_ref, base_ref, w1_ref, b1_ref, g0_ref, beta0_ref,
              g1_ref, beta1_ref, out_ref, z_ref):
  nt = (N * KNN) // TT
  qt = TT // KNN
  gm = _group_mat()
  cnt = 16.0 * N * KNN

  def tile_y(t):
    gt = g_ref[0, pl.ds(t * TT, TT), :]
    bt = base_ref[0, pl.ds(t * qt, qt), :]
    y = gt.reshape(qt, KNN, C) + bt[:, None, :]
    return y.reshape(TT, C)

  def p1(t, carry):
    s, q = carry
    y = tile_y(t)
    return (s + jnp.sum(y, axis=0, keepdims=True),
            q + jnp.sum(y * y, axis=0, keepdims=True))

  z1 = jnp.zeros((1, C), jnp.float32)
  s0, q0 = lax.fori_loop(0, nt, p1, (z1, z1))
  mean0 = jnp.dot(s0, gm, preferred_element_type=jnp.float32,
                      precision=lax.Precision.HIGHEST) / cnt
  var0 = jnp.dot(q0, gm, preferred_element_type=jnp.float32,
                      precision=lax.Precision.HIGHEST) / cnt - mean0 * mean0
  inv0 = lax.rsqrt(var0 + EPS)
  sc0 = inv0 * g0_ref[...]
  sh0 = beta0_ref[...] - mean0 * sc0

  w1 = w1_ref[...]
  b1 = b1_ref[...]

  def p2(t, carry):
    s, q = carry
    ya = tile_y(t) * sc0 + sh0
    ya = jnp.where(ya >= 0, ya, NEG_SLOPE * ya)
    z = lax.dot_general(ya, w1, (((1,), (1,)), ((), ())),
                        preferred_element_type=jnp.float32,
                      precision=lax.Precision.HIGHEST) + b1
    z_ref[pl.ds(t * TT, TT), :] = z
    return (s + jnp.sum(z, axis=0, keepdims=True),
            q + jnp.sum(z * z, axis=0, keepdims=True))

  s1, q1 = lax.fori_loop(0, nt, p2, (z1, z1))
  mean1 = jnp.dot(s1, gm, preferred_element_type=jnp.float32,
                      precision=lax.Precision.HIGHEST) / cnt
  var1 = jnp.dot(q1, gm, preferred_element_type=jnp.float32,
                      precision=lax.Precision.HIGHEST) / cnt - mean1 * mean1
  inv1 = lax.rsqrt(var1 + EPS)
  sc1 = inv1 * g1_ref[...]
  sh1 = beta1_ref[...] - mean1 * sc1

  def p3(t, carry):
    z = z_ref[pl.ds(t * TT, TT), :]
    za = z * sc1 + sh1
    za = jnp.where(za >= 0, za, NEG_SLOPE * za)
    zm = jnp.max(za.reshape(qt, KNN, C), axis=1)
    out_ref[0, :, pl.ds(t * qt, qt)] = zm.T
    return carry

  lax.fori_loop(0, nt, p3, 0)


def _mlp_call(g, base, w1, b1_2d, g0_2d, beta0_2d, g1_2d, beta1_2d):
  return pl.pallas_call(
      _mlp_body,
      grid=(B,),
      in_specs=[
          pl.BlockSpec((1, N * KNN, C), lambda b: (b, 0, 0)),
          pl.BlockSpec((1, N, C), lambda b: (b, 0, 0)),
          pl.BlockSpec((C, C), lambda b: (0, 0)),
          pl.BlockSpec((1, C), lambda b: (0, 0)),
          pl.BlockSpec((1, C), lambda b: (0, 0)),
          pl.BlockSpec((1, C), lambda b: (0, 0)),
          pl.BlockSpec((1, C), lambda b: (0, 0)),
          pl.BlockSpec((1, C), lambda b: (0, 0)),
      ],
      out_specs=pl.BlockSpec((1, C, N), lambda b: (b, 0, 0)),
      out_shape=jax.ShapeDtypeStruct((B, C, N), jnp.float32),
      scratch_shapes=[pltpu.VMEM((N * KNN, C), jnp.float32)],
      compiler_params=pltpu.CompilerParams(
          dimension_semantics=("arbitrary",)),
  )(g, base, w1, b1_2d, g0_2d, beta0_2d, g1_2d, beta1_2d)


def kernel(xyz1, xyz2, feat1, feat2, W0, b0, g0, beta0, W1, b1, g1, beta1):
  idx, base, pc = _knn_proj_call(xyz1, xyz2, feat1, feat2, W0,
                                 b0.reshape(1, C))
  gathered = _get_sc_gather()(pc.reshape(B * N2, C), idx.reshape(TOTAL_ROWS))
  return _mlp_call(gathered.reshape(B, N * KNN, C), base, W1,
                   b1.reshape(1, C), g0.reshape(1, C), beta0.reshape(1, C),
                   g1.reshape(1, C), beta1.reshape(1, C))


# speedup vs baseline: 9.1178x; 1.4728x over previous
import functools

import jax
import jax.numpy as jnp
from jax import lax
from jax.experimental import pallas as pl
from jax.experimental.pallas import tpu as pltpu
from jax.experimental.pallas import tpu_sc as plsc

KNN = 16
B, N, N2 = 8, 2048, 2048
C = 64
TI = 256
TT = 4096
EPS = 1e-5
NEG_SLOPE = 0.1

SC_WORKERS = 32
TOTAL_ROWS = B * N * KNN
ROWS_PER_W = TOTAL_ROWS // SC_WORKERS
SC_CHUNK = 128
SC_NCHUNK = ROWS_PER_W // SC_CHUNK


def _knn_proj_body(xyz1_ref, xyz2_ref, feat1_ref, feat2_ref, w0_ref, b0_ref,
                   idx_ref, base_ref, pc_ref, s_ref):
  b = pl.program_id(0)
  it = pl.program_id(1)
  x1 = xyz1_ref[0]
  x2 = xyz2_ref[0]
  w0 = w0_ref[...]

  n2 = jnp.sum(x2 * x2, axis=0, keepdims=True)
  g = lax.dot_general(x1, x2, (((0,), (0,)), ((), ())),
                      preferred_element_type=jnp.float32,
                      precision=lax.Precision.HIGHEST)
  s_ref[...] = n2 - 2.0 * g

  lane = lax.broadcasted_iota(jnp.int32, (8, 128), 1)
  nvr = N2 // 128
  inf = jnp.float32(jnp.inf)

  def topk8(sb):
    m = functools.reduce(jnp.minimum, sb)
    cols = []
    for r in range(KNN):
      gv = jnp.min(m, axis=1, keepdims=True)
      mparts, jparts = [], []
      for g0 in range(0, nvr, 4):
        mp, jp = [], []
        for v in range(g0, g0 + 4):
          emv = sb[v] == gv
          jp.append(jnp.where(emv, lane + v * 128, N2))
          if r < KNN - 1:
            sb[v] = jnp.where(emv, inf, sb[v])
            mp.append(sb[v])
        jparts.append(jnp.minimum(jnp.minimum(jp[0], jp[1]),
                                  jnp.minimum(jp[2], jp[3])))
        if r < KNN - 1:
          mparts.append(jnp.minimum(jnp.minimum(mp[0], mp[1]),
                                    jnp.minimum(mp[2], mp[3])))
      if r < KNN - 1:
        m = jnp.minimum(jnp.minimum(mparts[0], mparts[1]),
                        jnp.minimum(mparts[2], mparts[3]))
      gj = functools.reduce(jnp.minimum, jparts)
      cols.append(jnp.min(gj, axis=1, keepdims=True))
    return jnp.concatenate(cols, axis=1)

  def blk(j, carry):
    for u in range(4):
      row = j * 32 + u * 8
      sbw = s_ref[pl.ds(row, 8), :]
      sb = [sbw[:, v * 128:(v + 1) * 128] for v in range(nvr)]
      idx_ref[0, pl.ds(row, 8), :] = topk8(sb) + b * N2
    return carry

  lax.fori_loop(0, TI // 32, blk, 0)

  f1 = feat1_ref[0]
  bt = (lax.dot_general(f1, w0[:, :C], (((0,), (1,)), ((), ())),
                        preferred_element_type=jnp.float32,
                      precision=lax.Precision.HIGHEST)
        - lax.dot_general(x1, w0[:, 2 * C:], (((0,), (1,)), ((), ())),
                          preferred_element_type=jnp.float32,
                      precision=lax.Precision.HIGHEST)
        + b0_ref[...])
  base_ref[0] = bt

  @pl.when(it == 0)
  def _():
    f2 = feat2_ref[0]
    pcv = (lax.dot_general(f2, w0[:, C:2 * C], (((0,), (1,)), ((), ())),
                           preferred_element_type=jnp.float32,
                      precision=lax.Precision.HIGHEST)
           + lax.dot_general(x2, w0[:, 2 * C:], (((0,), (1,)), ((), ())),
                             preferred_element_type=jnp.float32,
                      precision=lax.Precision.HIGHEST))
    pc_ref[0] = pcv


def _knn_proj_call(xyz1, xyz2, feat1, feat2, w0, b0_2d):
  return pl.pallas_call(
      _knn_proj_body,
      grid=(B, N // TI),
      in_specs=[
          pl.BlockSpec((1, 3, TI), lambda b, it: (b, 0, it)),
          pl.BlockSpec((1, 3, N2), lambda b, it: (b, 0, 0)),
          pl.BlockSpec((1, C, TI), lambda b, it: (b, 0, it)),
          pl.BlockSpec((1, C, N2), lambda b, it: (b, 0, 0)),
          pl.BlockSpec((C, 131), lambda b, it: (0, 0)),
          pl.BlockSpec((1, C), lambda b, it: (0, 0)),
      ],
      out_specs=[
          pl.BlockSpec((1, TI, KNN), lambda b, it: (b, it, 0)),
          pl.BlockSpec((1, TI, C), lambda b, it: (b, it, 0)),
          pl.BlockSpec((1, N2, C), lambda b, it: (b, 0, 0)),
      ],
      out_shape=[
          jax.ShapeDtypeStruct((B, N, KNN), jnp.int32),
          jax.ShapeDtypeStruct((B, N, C), jnp.float32),
          jax.ShapeDtypeStruct((B, N2, C), jnp.float32),
      ],
      scratch_shapes=[pltpu.VMEM((TI, N2), jnp.float32)],
      compiler_params=pltpu.CompilerParams(
          dimension_semantics=("arbitrary", "arbitrary")),
  )(xyz1, xyz2, feat1, feat2, w0, b0_2d)


def _sc_gather_body(table_hbm, idx_hbm, out_hbm, idx_v, rows_v, sem):
  wid = lax.axis_index("s") * 2 + lax.axis_index("c")
  base = wid * ROWS_PER_W

  def chunk(ci, carry):
    off = base + ci * SC_CHUNK
    pltpu.sync_copy(idx_hbm.at[pl.ds(off, SC_CHUNK)], idx_v)
    pltpu.async_copy(table_hbm.at[idx_v], rows_v, sem).wait()
    pltpu.sync_copy(rows_v, out_hbm.at[pl.ds(off, SC_CHUNK)])
    return carry

  lax.fori_loop(0, SC_NCHUNK, chunk, 0)


@functools.cache
def _get_sc_gather():
  return pl.kernel(
      _sc_gather_body,
      out_type=jax.ShapeDtypeStruct((TOTAL_ROWS, C), jnp.float32),
      mesh=plsc.VectorSubcoreMesh(core_axis_name="c", subcore_axis_name="s"),
      scratch_types=[
          pltpu.VMEM((SC_CHUNK,), jnp.int32),
          pltpu.VMEM((SC_CHUNK, C), jnp.float32),
          pltpu.SemaphoreType.DMA,
      ],
      compiler_params=pltpu.CompilerParams(use_tc_tiling_on_sc=False),
  )


def _group_mat():
  i = lax.broadcasted_iota(jnp.int32, (C, C), 0)
  j = lax.broadcasted_iota(jnp.int32, (C, C), 1)
  return ((i // 16) == (j // 16)).astype(jnp.float32)


def _mlp_body(g_ref, base_ref, w1_ref, b1_ref, g0_ref, beta0_ref,
              g1_ref, beta1_ref, out_ref, z_ref):
  nt = (N * KNN) // TT
  qt = TT // KNN
  gm = _group_mat()
  cnt = 16.0 * N * KNN

  def tile_y(t):
    gt = g_ref[0, pl.ds(t * TT, TT), :]
    bt = base_ref[0, pl.ds(t * qt, qt), :]
    y = gt.reshape(qt, KNN, C) + bt[:, None, :]
    return y.reshape(TT, C)

  def p1(t, carry):
    s, q = carry
    y = tile_y(t)
    return (s + jnp.sum(y, axis=0, keepdims=True),
            q + jnp.sum(y * y, axis=0, keepdims=True))

  z1 = jnp.zeros((1, C), jnp.float32)
  s0, q0 = lax.fori_loop(0, nt, p1, (z1, z1))
  mean0 = jnp.dot(s0, gm, preferred_element_type=jnp.float32,
                      precision=lax.Precision.HIGHEST) / cnt
  var0 = jnp.dot(q0, gm, preferred_element_type=jnp.float32,
                      precision=lax.Precision.HIGHEST) / cnt - mean0 * mean0
  inv0 = lax.rsqrt(var0 + EPS)
  sc0 = inv0 * g0_ref[...]
  sh0 = beta0_ref[...] - mean0 * sc0

  w1 = w1_ref[...]
  b1 = b1_ref[...]

  def p2(t, carry):
    s, q = carry
    ya = tile_y(t) * sc0 + sh0
    ya = jnp.where(ya >= 0, ya, NEG_SLOPE * ya)
    z = lax.dot_general(ya, w1, (((1,), (1,)), ((), ())),
                        preferred_element_type=jnp.float32,
                      precision=lax.Precision.HIGHEST) + b1
    z_ref[pl.ds(t * TT, TT), :] = z
    return (s + jnp.sum(z, axis=0, keepdims=True),
            q + jnp.sum(z * z, axis=0, keepdims=True))

  s1, q1 = lax.fori_loop(0, nt, p2, (z1, z1))
  mean1 = jnp.dot(s1, gm, preferred_element_type=jnp.float32,
                      precision=lax.Precision.HIGHEST) / cnt
  var1 = jnp.dot(q1, gm, preferred_element_type=jnp.float32,
                      precision=lax.Precision.HIGHEST) / cnt - mean1 * mean1
  inv1 = lax.rsqrt(var1 + EPS)
  sc1 = inv1 * g1_ref[...]
  sh1 = beta1_ref[...] - mean1 * sc1

  def p3(t, carry):
    z = z_ref[pl.ds(t * TT, TT), :]
    za = z * sc1 + sh1
    za = jnp.where(za >= 0, za, NEG_SLOPE * za)
    zm = jnp.max(za.reshape(qt, KNN, C), axis=1)
    out_ref[0, :, pl.ds(t * qt, qt)] = zm.T
    return carry

  lax.fori_loop(0, nt, p3, 0)


def _mlp_call(g, base, w1, b1_2d, g0_2d, beta0_2d, g1_2d, beta1_2d):
  return pl.pallas_call(
      _mlp_body,
      grid=(B,),
      in_specs=[
          pl.BlockSpec((1, N * KNN, C), lambda b: (b, 0, 0)),
          pl.BlockSpec((1, N, C), lambda b: (b, 0, 0)),
          pl.BlockSpec((C, C), lambda b: (0, 0)),
          pl.BlockSpec((1, C), lambda b: (0, 0)),
          pl.BlockSpec((1, C), lambda b: (0, 0)),
          pl.BlockSpec((1, C), lambda b: (0, 0)),
          pl.BlockSpec((1, C), lambda b: (0, 0)),
          pl.BlockSpec((1, C), lambda b: (0, 0)),
      ],
      out_specs=pl.BlockSpec((1, C, N), lambda b: (b, 0, 0)),
      out_shape=jax.ShapeDtypeStruct((B, C, N), jnp.float32),
      scratch_shapes=[pltpu.VMEM((N * KNN, C), jnp.float32)],
      compiler_params=pltpu.CompilerParams(
          dimension_semantics=("arbitrary",)),
  )(g, base, w1, b1_2d, g0_2d, beta0_2d, g1_2d, beta1_2d)


def kernel(xyz1, xyz2, feat1, feat2, W0, b0, g0, beta0, W1, b1, g1, beta1):
  idx, base, pc = _knn_proj_call(xyz1, xyz2, feat1, feat2, W0,
                                 b0.reshape(1, C))
  gathered = _get_sc_gather()(pc.reshape(B * N2, C), idx.reshape(TOTAL_ROWS))
  return _mlp_call(gathered.reshape(B, N * KNN, C), base, W1,
                   b1.reshape(1, C), g0.reshape(1, C), beta0.reshape(1, C),
                   g1.reshape(1, C), beta1.reshape(1, C))


# speedup vs baseline: 11.1223x; 1.2198x over previous
import functools

import jax
import jax.numpy as jnp
from jax import lax
from jax.experimental import pallas as pl
from jax.experimental.pallas import tpu as pltpu
from jax.experimental.pallas import tpu_sc as plsc

KNN = 16
B, N, N2 = 8, 2048, 2048
C = 64
TI = 256
TT = 4096
EPS = 1e-5
NEG_SLOPE = 0.1

SC_WORKERS = 32
TOTAL_ROWS = B * N * KNN
ROWS_PER_W = TOTAL_ROWS // SC_WORKERS
SC_CHUNK = 128
SC_NCHUNK = ROWS_PER_W // SC_CHUNK


def _knn_proj_body(xyz1_ref, xyz2_ref, feat1_ref, feat2_ref, w0_ref, b0_ref,
                   idx_ref, base_ref, pc_ref, s_ref):
  b = pl.program_id(0)
  it = pl.program_id(1)
  x1 = xyz1_ref[0]
  x2 = xyz2_ref[0]
  w0 = w0_ref[...]

  n2 = jnp.sum(x2 * x2, axis=0, keepdims=True)
  g = lax.dot_general(x1, x2, (((0,), (0,)), ((), ())),
                      preferred_element_type=jnp.float32,
                      precision=lax.Precision.HIGHEST)
  s_ref[...] = n2 - 2.0 * g

  lane = lax.broadcasted_iota(jnp.int32, (8, 128), 1)
  nvr = N2 // 128
  inf = jnp.float32(jnp.inf)

  def topk8(sb):
    m = functools.reduce(jnp.minimum, sb)
    cols = []
    for r in range(KNN):
      gv = jnp.min(m, axis=1, keepdims=True)
      mparts, jparts = [], []
      for g0 in range(0, nvr, 4):
        mp, jp = [], []
        for v in range(g0, g0 + 4):
          emv = sb[v] == gv
          jp.append(jnp.where(emv, lane + v * 128, N2))
          if r < KNN - 1:
            sb[v] = jnp.where(emv, inf, sb[v])
            mp.append(sb[v])
        jparts.append(jnp.minimum(jnp.minimum(jp[0], jp[1]),
                                  jnp.minimum(jp[2], jp[3])))
        if r < KNN - 1:
          mparts.append(jnp.minimum(jnp.minimum(mp[0], mp[1]),
                                    jnp.minimum(mp[2], mp[3])))
      if r < KNN - 1:
        m = jnp.minimum(jnp.minimum(mparts[0], mparts[1]),
                        jnp.minimum(mparts[2], mparts[3]))
      gj = functools.reduce(jnp.minimum, jparts)
      cols.append(jnp.min(gj, axis=1, keepdims=True))
    return jnp.concatenate(cols, axis=1)

  def blk(j, carry):
    for u in range(8):
      row = j * 64 + u * 8
      sbw = s_ref[pl.ds(row, 8), :]
      sb = [sbw[:, v * 128:(v + 1) * 128] for v in range(nvr)]
      idx_ref[0, pl.ds(row, 8), :] = topk8(sb) + b * N2
    return carry

  lax.fori_loop(0, TI // 64, blk, 0)

  f1 = feat1_ref[0]
  bt = (lax.dot_general(f1, w0[:, :C], (((0,), (1,)), ((), ())),
                        preferred_element_type=jnp.float32,
                      precision=lax.Precision.HIGHEST)
        - lax.dot_general(x1, w0[:, 2 * C:], (((0,), (1,)), ((), ())),
                          preferred_element_type=jnp.float32,
                      precision=lax.Precision.HIGHEST)
        + b0_ref[...])
  base_ref[0] = bt

  @pl.when(it == 0)
  def _():
    f2 = feat2_ref[0]
    pcv = (lax.dot_general(f2, w0[:, C:2 * C], (((0,), (1,)), ((), ())),
                           preferred_element_type=jnp.float32,
                      precision=lax.Precision.HIGHEST)
           + lax.dot_general(x2, w0[:, 2 * C:], (((0,), (1,)), ((), ())),
                             preferred_element_type=jnp.float32,
                      precision=lax.Precision.HIGHEST))
    pc_ref[0] = pcv


def _knn_proj_call(xyz1, xyz2, feat1, feat2, w0, b0_2d):
  return pl.pallas_call(
      _knn_proj_body,
      grid=(B, N // TI),
      in_specs=[
          pl.BlockSpec((1, 3, TI), lambda b, it: (b, 0, it)),
          pl.BlockSpec((1, 3, N2), lambda b, it: (b, 0, 0)),
          pl.BlockSpec((1, C, TI), lambda b, it: (b, 0, it)),
          pl.BlockSpec((1, C, N2), lambda b, it: (b, 0, 0)),
          pl.BlockSpec((C, 131), lambda b, it: (0, 0)),
          pl.BlockSpec((1, C), lambda b, it: (0, 0)),
      ],
      out_specs=[
          pl.BlockSpec((1, TI, KNN), lambda b, it: (b, it, 0)),
          pl.BlockSpec((1, TI, C), lambda b, it: (b, it, 0)),
          pl.BlockSpec((1, N2, C), lambda b, it: (b, 0, 0)),
      ],
      out_shape=[
          jax.ShapeDtypeStruct((B, N, KNN), jnp.int32),
          jax.ShapeDtypeStruct((B, N, C), jnp.float32),
          jax.ShapeDtypeStruct((B, N2, C), jnp.float32),
      ],
      scratch_shapes=[pltpu.VMEM((TI, N2), jnp.float32)],
      compiler_params=pltpu.CompilerParams(
          dimension_semantics=("arbitrary", "arbitrary")),
  )(xyz1, xyz2, feat1, feat2, w0, b0_2d)


def _sc_gather_body(table_hbm, idx_hbm, out_hbm, idx_v, rows_v, sem):
  wid = lax.axis_index("s") * 2 + lax.axis_index("c")
  base = wid * ROWS_PER_W

  def chunk(ci, carry):
    off = base + ci * SC_CHUNK
    pltpu.sync_copy(idx_hbm.at[pl.ds(off, SC_CHUNK)], idx_v)
    pltpu.async_copy(table_hbm.at[idx_v], rows_v, sem).wait()
    pltpu.sync_copy(rows_v, out_hbm.at[pl.ds(off, SC_CHUNK)])
    return carry

  lax.fori_loop(0, SC_NCHUNK, chunk, 0)


@functools.cache
def _get_sc_gather():
  return pl.kernel(
      _sc_gather_body,
      out_type=jax.ShapeDtypeStruct((TOTAL_ROWS, C), jnp.float32),
      mesh=plsc.VectorSubcoreMesh(core_axis_name="c", subcore_axis_name="s"),
      scratch_types=[
          pltpu.VMEM((SC_CHUNK,), jnp.int32),
          pltpu.VMEM((SC_CHUNK, C), jnp.float32),
          pltpu.SemaphoreType.DMA,
      ],
      compiler_params=pltpu.CompilerParams(use_tc_tiling_on_sc=False),
  )


def _group_mat():
  i = lax.broadcasted_iota(jnp.int32, (C, C), 0)
  j = lax.broadcasted_iota(jnp.int32, (C, C), 1)
  return ((i // 16) == (j // 16)).astype(jnp.float32)


def _mlp_body(g_ref, base_ref, w1_ref, b1_ref, g0_ref, beta0_ref,
              g1_ref, beta1_ref, out_ref, z_ref):
  nt = (N * KNN) // TT
  qt = TT // KNN
  gm = _group_mat()
  cnt = 16.0 * N * KNN

  def tile_y(t):
    gt = g_ref[0, pl.ds(t * TT, TT), :]
    bt = base_ref[0, pl.ds(t * qt, qt), :]
    y = gt.reshape(qt, KNN, C) + bt[:, None, :]
    return y.reshape(TT, C)

  def p1(t, carry):
    s, q = carry
    y = tile_y(t)
    return (s + jnp.sum(y, axis=0, keepdims=True),
            q + jnp.sum(y * y, axis=0, keepdims=True))

  z1 = jnp.zeros((1, C), jnp.float32)
  s0, q0 = lax.fori_loop(0, nt, p1, (z1, z1))
  mean0 = jnp.dot(s0, gm, preferred_element_type=jnp.float32,
                      precision=lax.Precision.HIGHEST) / cnt
  var0 = jnp.dot(q0, gm, preferred_element_type=jnp.float32,
                      precision=lax.Precision.HIGHEST) / cnt - mean0 * mean0
  inv0 = lax.rsqrt(var0 + EPS)
  sc0 = inv0 * g0_ref[...]
  sh0 = beta0_ref[...] - mean0 * sc0

  w1 = w1_ref[...]
  b1 = b1_ref[...]

  def p2(t, carry):
    s, q = carry
    ya = tile_y(t) * sc0 + sh0
    ya = jnp.where(ya >= 0, ya, NEG_SLOPE * ya)
    z = lax.dot_general(ya, w1, (((1,), (1,)), ((), ())),
                        preferred_element_type=jnp.float32,
                      precision=lax.Precision.HIGHEST) + b1
    z_ref[pl.ds(t * TT, TT), :] = z
    return (s + jnp.sum(z, axis=0, keepdims=True),
            q + jnp.sum(z * z, axis=0, keepdims=True))

  s1, q1 = lax.fori_loop(0, nt, p2, (z1, z1))
  mean1 = jnp.dot(s1, gm, preferred_element_type=jnp.float32,
                      precision=lax.Precision.HIGHEST) / cnt
  var1 = jnp.dot(q1, gm, preferred_element_type=jnp.float32,
                      precision=lax.Precision.HIGHEST) / cnt - mean1 * mean1
  inv1 = lax.rsqrt(var1 + EPS)
  sc1 = inv1 * g1_ref[...]
  sh1 = beta1_ref[...] - mean1 * sc1

  def p3(t, carry):
    z = z_ref[pl.ds(t * TT, TT), :]
    za = z * sc1 + sh1
    za = jnp.where(za >= 0, za, NEG_SLOPE * za)
    zm = jnp.max(za.reshape(qt, KNN, C), axis=1)
    out_ref[0, :, pl.ds(t * qt, qt)] = zm.T
    return carry

  lax.fori_loop(0, nt, p3, 0)


def _mlp_call(g, base, w1, b1_2d, g0_2d, beta0_2d, g1_2d, beta1_2d):
  return pl.pallas_call(
      _mlp_body,
      grid=(B,),
      in_specs=[
          pl.BlockSpec((1, N * KNN, C), lambda b: (b, 0, 0)),
          pl.BlockSpec((1, N, C), lambda b: (b, 0, 0)),
          pl.BlockSpec((C, C), lambda b: (0, 0)),
          pl.BlockSpec((1, C), lambda b: (0, 0)),
          pl.BlockSpec((1, C), lambda b: (0, 0)),
          pl.BlockSpec((1, C), lambda b: (0, 0)),
          pl.BlockSpec((1, C), lambda b: (0, 0)),
          pl.BlockSpec((1, C), lambda b: (0, 0)),
      ],
      out_specs=pl.BlockSpec((1, C, N), lambda b: (b, 0, 0)),
      out_shape=jax.ShapeDtypeStruct((B, C, N), jnp.float32),
      scratch_shapes=[pltpu.VMEM((N * KNN, C), jnp.float32)],
      compiler_params=pltpu.CompilerParams(
          dimension_semantics=("arbitrary",)),
  )(g, base, w1, b1_2d, g0_2d, beta0_2d, g1_2d, beta1_2d)


def kernel(xyz1, xyz2, feat1, feat2, W0, b0, g0, beta0, W1, b1, g1, beta1):
  idx, base, pc = _knn_proj_call(xyz1, xyz2, feat1, feat2, W0,
                                 b0.reshape(1, C))
  gathered = _get_sc_gather()(pc.reshape(B * N2, C), idx.reshape(TOTAL_ROWS))
  return _mlp_call(gathered.reshape(B, N * KNN, C), base, W1,
                   b1.reshape(1, C), g0.reshape(1, C), beta0.reshape(1, C),
                   g1.reshape(1, C), beta1.reshape(1, C))


# speedup vs baseline: 14.4241x; 1.2969x over previous
import functools

import jax
import jax.numpy as jnp
from jax import lax
from jax.experimental import pallas as pl
from jax.experimental.pallas import tpu as pltpu
from jax.experimental.pallas import tpu_sc as plsc

KNN = 16
B, N, N2 = 8, 2048, 2048
C = 64
TI = 256
TT = 4096
EPS = 1e-5
NEG_SLOPE = 0.1

SC_WORKERS = 32
TOTAL_ROWS = B * N * KNN
ROWS_PER_W = TOTAL_ROWS // SC_WORKERS
SC_CHUNK = 128
SC_NCHUNK = ROWS_PER_W // SC_CHUNK


def _knn_proj_body(xyz1_ref, xyz2_ref, feat1_ref, feat2_ref, w0_ref, b0_ref,
                   idx_ref, base_ref, pc_ref, s_ref):
  b = pl.program_id(0)
  it = pl.program_id(1)
  x1 = xyz1_ref[0]
  x2 = xyz2_ref[0]
  w0 = w0_ref[...]

  n2 = jnp.sum(x2 * x2, axis=0, keepdims=True)
  g = lax.dot_general(x1, x2, (((0,), (0,)), ((), ())),
                      preferred_element_type=jnp.float32,
                      precision=lax.Precision.HIGHEST)
  s_ref[...] = n2 - 2.0 * g

  lane = lax.broadcasted_iota(jnp.int32, (8, 128), 1)
  nvr = N2 // 128
  inf = jnp.float32(jnp.inf)

  def topk8(sb):
    m = functools.reduce(jnp.minimum, sb)
    cols = []
    for r in range(KNN):
      gv = jnp.min(m, axis=1, keepdims=True)
      mparts, jparts = [], []
      for g0 in range(0, nvr, 4):
        mp, jp = [], []
        for v in range(g0, g0 + 4):
          emv = sb[v] == gv
          jp.append(jnp.where(emv, lane + v * 128, N2))
          if r < KNN - 1:
            sb[v] = jnp.where(emv, inf, sb[v])
            mp.append(sb[v])
        jparts.append(jnp.minimum(jnp.minimum(jp[0], jp[1]),
                                  jnp.minimum(jp[2], jp[3])))
        if r < KNN - 1:
          mparts.append(jnp.minimum(jnp.minimum(mp[0], mp[1]),
                                    jnp.minimum(mp[2], mp[3])))
      if r < KNN - 1:
        m = jnp.minimum(jnp.minimum(mparts[0], mparts[1]),
                        jnp.minimum(mparts[2], mparts[3]))
      gj = functools.reduce(jnp.minimum, jparts)
      cols.append(jnp.min(gj, axis=1, keepdims=True))
    return jnp.concatenate(cols, axis=1)

  def blk(j, carry):
    for u in range(16):
      row = j * 128 + u * 8
      sbw = s_ref[pl.ds(row, 8), :]
      sb = [sbw[:, v * 128:(v + 1) * 128] for v in range(nvr)]
      idx_ref[0, pl.ds(row, 8), :] = topk8(sb) + b * N2
    return carry

  lax.fori_loop(0, TI // 128, blk, 0)

  f1 = feat1_ref[0]
  bt = (lax.dot_general(f1, w0[:, :C], (((0,), (1,)), ((), ())),
                        preferred_element_type=jnp.float32,
                      precision=lax.Precision.HIGHEST)
        - lax.dot_general(x1, w0[:, 2 * C:], (((0,), (1,)), ((), ())),
                          preferred_element_type=jnp.float32,
                      precision=lax.Precision.HIGHEST)
        + b0_ref[...])
  base_ref[0] = bt

  @pl.when(it == 0)
  def _():
    f2 = feat2_ref[0]
    pcv = (lax.dot_general(f2, w0[:, C:2 * C], (((0,), (1,)), ((), ())),
                           preferred_element_type=jnp.float32,
                      precision=lax.Precision.HIGHEST)
           + lax.dot_general(x2, w0[:, 2 * C:], (((0,), (1,)), ((), ())),
                             preferred_element_type=jnp.float32,
                      precision=lax.Precision.HIGHEST))
    pc_ref[0] = pcv


def _knn_proj_call(xyz1, xyz2, feat1, feat2, w0, b0_2d):
  return pl.pallas_call(
      _knn_proj_body,
      grid=(B, N // TI),
      in_specs=[
          pl.BlockSpec((1, 3, TI), lambda b, it: (b, 0, it)),
          pl.BlockSpec((1, 3, N2), lambda b, it: (b, 0, 0)),
          pl.BlockSpec((1, C, TI), lambda b, it: (b, 0, it)),
          pl.BlockSpec((1, C, N2), lambda b, it: (b, 0, 0)),
          pl.BlockSpec((C, 131), lambda b, it: (0, 0)),
          pl.BlockSpec((1, C), lambda b, it: (0, 0)),
      ],
      out_specs=[
          pl.BlockSpec((1, TI, KNN), lambda b, it: (b, it, 0)),
          pl.BlockSpec((1, TI, C), lambda b, it: (b, it, 0)),
          pl.BlockSpec((1, N2, C), lambda b, it: (b, 0, 0)),
      ],
      out_shape=[
          jax.ShapeDtypeStruct((B, N, KNN), jnp.int32),
          jax.ShapeDtypeStruct((B, N, C), jnp.float32),
          jax.ShapeDtypeStruct((B, N2, C), jnp.float32),
      ],
      scratch_shapes=[pltpu.VMEM((TI, N2), jnp.float32)],
      compiler_params=pltpu.CompilerParams(
          dimension_semantics=("arbitrary", "arbitrary")),
  )(xyz1, xyz2, feat1, feat2, w0, b0_2d)


def _sc_gather_body(table_hbm, idx_hbm, out_hbm, idx_v, rows_v, sem):
  wid = lax.axis_index("s") * 2 + lax.axis_index("c")
  base = wid * ROWS_PER_W

  def chunk(ci, carry):
    off = base + ci * SC_CHUNK
    pltpu.sync_copy(idx_hbm.at[pl.ds(off, SC_CHUNK)], idx_v)
    pltpu.async_copy(table_hbm.at[idx_v], rows_v, sem).wait()
    pltpu.sync_copy(rows_v, out_hbm.at[pl.ds(off, SC_CHUNK)])
    return carry

  lax.fori_loop(0, SC_NCHUNK, chunk, 0)


@functools.cache
def _get_sc_gather():
  return pl.kernel(
      _sc_gather_body,
      out_type=jax.ShapeDtypeStruct((TOTAL_ROWS, C), jnp.float32),
      mesh=plsc.VectorSubcoreMesh(core_axis_name="c", subcore_axis_name="s"),
      scratch_types=[
          pltpu.VMEM((SC_CHUNK,), jnp.int32),
          pltpu.VMEM((SC_CHUNK, C), jnp.float32),
          pltpu.SemaphoreType.DMA,
      ],
      compiler_params=pltpu.CompilerParams(use_tc_tiling_on_sc=False),
  )


def _group_mat():
  i = lax.broadcasted_iota(jnp.int32, (C, C), 0)
  j = lax.broadcasted_iota(jnp.int32, (C, C), 1)
  return ((i // 16) == (j // 16)).astype(jnp.float32)


def _mlp_body(g_ref, base_ref, w1_ref, b1_ref, g0_ref, beta0_ref,
              g1_ref, beta1_ref, out_ref, z_ref):
  rows = N * KNN // 2
  tt = 2048
  qt = tt // (KNN // 2)
  nt = rows // tt
  gm = _group_mat()
  cnt = 16.0 * N * KNN

  def pair(x):
    return jnp.concatenate([x, x], axis=1)

  def tile_y(t):
    gt = g_ref[0, pl.ds(t * tt, tt), :]
    bt = base_ref[0, pl.ds(t * qt, qt), :]
    bp = jnp.concatenate([bt, bt], axis=1)
    y = gt.reshape(qt, KNN // 2, 2 * C) + bp[:, None, :]
    return y.reshape(tt, 2 * C)

  def fold(s):
    return s[:, :C] + s[:, C:]

  def p1(t, carry):
    s, q = carry
    y = tile_y(t)
    return (s + jnp.sum(y, axis=0, keepdims=True),
            q + jnp.sum(y * y, axis=0, keepdims=True))

  z1 = jnp.zeros((1, 2 * C), jnp.float32)
  s0, q0 = lax.fori_loop(0, nt, p1, (z1, z1))
  mean0 = jnp.dot(fold(s0), gm, preferred_element_type=jnp.float32,
                  precision=lax.Precision.HIGHEST) / cnt
  var0 = jnp.dot(fold(q0), gm, preferred_element_type=jnp.float32,
                 precision=lax.Precision.HIGHEST) / cnt - mean0 * mean0
  inv0 = lax.rsqrt(var0 + EPS)
  sc0 = pair(inv0 * g0_ref[...])
  sh0 = pair(beta0_ref[...] - mean0 * inv0 * g0_ref[...])

  w1 = w1_ref[...]
  zc = jnp.zeros((C, C), jnp.float32)
  w2 = jnp.concatenate([jnp.concatenate([w1, zc], axis=1),
                        jnp.concatenate([zc, w1], axis=1)], axis=0)
  b1p = pair(b1_ref[...])

  def p2(t, carry):
    s, q = carry
    ya = tile_y(t) * sc0 + sh0
    ya = jnp.where(ya >= 0, ya, NEG_SLOPE * ya)
    z = lax.dot_general(ya, w2, (((1,), (1,)), ((), ())),
                        preferred_element_type=jnp.float32,
                        precision=lax.Precision.HIGHEST) + b1p
    z_ref[pl.ds(t * tt, tt), :] = z
    return (s + jnp.sum(z, axis=0, keepdims=True),
            q + jnp.sum(z * z, axis=0, keepdims=True))

  s1, q1 = lax.fori_loop(0, nt, p2, (z1, z1))
  mean1 = jnp.dot(fold(s1), gm, preferred_element_type=jnp.float32,
                  precision=lax.Precision.HIGHEST) / cnt
  var1 = jnp.dot(fold(q1), gm, preferred_element_type=jnp.float32,
                 precision=lax.Precision.HIGHEST) / cnt - mean1 * mean1
  inv1 = lax.rsqrt(var1 + EPS)
  sc1 = pair(inv1 * g1_ref[...])
  sh1 = pair(beta1_ref[...] - mean1 * inv1 * g1_ref[...])

  def p3(t, carry):
    z = z_ref[pl.ds(t * tt, tt), :]
    za = z * sc1 + sh1
    za = jnp.where(za >= 0, za, NEG_SLOPE * za)
    zm = jnp.max(za.reshape(qt, KNN // 2, 2 * C), axis=1)
    zq = jnp.maximum(zm[:, :C], zm[:, C:])
    out_ref[0, :, pl.ds(t * qt, qt)] = zq.T
    return carry

  lax.fori_loop(0, nt, p3, 0)


def _mlp_call(g2, base, w1, b1_2d, g0_2d, beta0_2d, g1_2d, beta1_2d):
  return pl.pallas_call(
      _mlp_body,
      grid=(B,),
      in_specs=[
          pl.BlockSpec((1, N * KNN // 2, 2 * C), lambda b: (b, 0, 0)),
          pl.BlockSpec((1, N, C), lambda b: (b, 0, 0)),
          pl.BlockSpec((C, C), lambda b: (0, 0)),
          pl.BlockSpec((1, C), lambda b: (0, 0)),
          pl.BlockSpec((1, C), lambda b: (0, 0)),
          pl.BlockSpec((1, C), lambda b: (0, 0)),
          pl.BlockSpec((1, C), lambda b: (0, 0)),
          pl.BlockSpec((1, C), lambda b: (0, 0)),
      ],
      out_specs=pl.BlockSpec((1, C, N), lambda b: (b, 0, 0)),
      out_shape=jax.ShapeDtypeStruct((B, C, N), jnp.float32),
      scratch_shapes=[pltpu.VMEM((N * KNN // 2, 2 * C), jnp.float32)],
      compiler_params=pltpu.CompilerParams(
          dimension_semantics=("arbitrary",)),
  )(g2, base, w1, b1_2d, g0_2d, beta0_2d, g1_2d, beta1_2d)


def kernel(xyz1, xyz2, feat1, feat2, W0, b0, g0, beta0, W1, b1, g1, beta1):
  idx, base, pc = _knn_proj_call(xyz1, xyz2, feat1, feat2, W0,
                                 b0.reshape(1, C))
  gathered = _get_sc_gather()(pc.reshape(B * N2, C), idx.reshape(TOTAL_ROWS))
  return _mlp_call(gathered.reshape(B, N * KNN // 2, 2 * C), base, W1,
                   b1.reshape(1, C), g0.reshape(1, C), beta0.reshape(1, C),
                   g1.reshape(1, C), beta1.reshape(1, C))


# speedup vs baseline: 15.5771x; 1.0799x over previous
import functools

import jax
import jax.numpy as jnp
from jax import lax
from jax.experimental import pallas as pl
from jax.experimental.pallas import tpu as pltpu
from jax.experimental.pallas import tpu_sc as plsc

KNN = 16
B, N, N2 = 8, 2048, 2048
C = 64
TI = 256
TT = 4096
EPS = 1e-5
NEG_SLOPE = 0.1

SC_WORKERS = 32
TOTAL_ROWS = B * N * KNN
ROWS_PER_W = TOTAL_ROWS // SC_WORKERS
SC_CHUNK = 128
SC_NCHUNK = ROWS_PER_W // SC_CHUNK
SC_GRP = 8


def _knn_proj_body(xyz1_ref, xyz2_ref, feat1_ref, feat2_ref, w0_ref, b0_ref,
                   idx_ref, base_ref, pc_ref, s_ref):
  b = pl.program_id(0)
  it = pl.program_id(1)
  x1 = xyz1_ref[0]
  x2 = xyz2_ref[0]
  w0 = w0_ref[...]

  n2 = jnp.sum(x2 * x2, axis=0, keepdims=True)
  g = lax.dot_general(x1, x2, (((0,), (0,)), ((), ())),
                      preferred_element_type=jnp.float32,
                      precision=lax.Precision.HIGHEST)
  s_ref[...] = n2 - 2.0 * g

  lane = lax.broadcasted_iota(jnp.int32, (8, 128), 1)
  nvr = N2 // 128
  inf = jnp.float32(jnp.inf)

  def topk8(sb):
    m = functools.reduce(jnp.minimum, sb)
    cols = []
    for r in range(KNN):
      gv = jnp.min(m, axis=1, keepdims=True)
      mparts, jparts = [], []
      for g0 in range(0, nvr, 4):
        mp, jp = [], []
        for v in range(g0, g0 + 4):
          emv = sb[v] == gv
          jp.append(jnp.where(emv, lane + v * 128, N2))
          if r < KNN - 1:
            sb[v] = jnp.where(emv, inf, sb[v])
            mp.append(sb[v])
        jparts.append(jnp.minimum(jnp.minimum(jp[0], jp[1]),
                                  jnp.minimum(jp[2], jp[3])))
        if r < KNN - 1:
          mparts.append(jnp.minimum(jnp.minimum(mp[0], mp[1]),
                                    jnp.minimum(mp[2], mp[3])))
      if r < KNN - 1:
        m = jnp.minimum(jnp.minimum(mparts[0], mparts[1]),
                        jnp.minimum(mparts[2], mparts[3]))
      gj = functools.reduce(jnp.minimum, jparts)
      cols.append(jnp.min(gj, axis=1, keepdims=True))
    return jnp.concatenate(cols, axis=1)

  def blk(j, carry):
    for u in range(16):
      row = j * 128 + u * 8
      sbw = s_ref[pl.ds(row, 8), :]
      sb = [sbw[:, v * 128:(v + 1) * 128] for v in range(nvr)]
      idx_ref[0, pl.ds(row, 8), :] = topk8(sb) + b * N2
    return carry

  lax.fori_loop(0, TI // 128, blk, 0)

  f1 = feat1_ref[0]
  bt = (lax.dot_general(f1, w0[:, :C], (((0,), (1,)), ((), ())),
                        preferred_element_type=jnp.float32,
                      precision=lax.Precision.HIGHEST)
        - lax.dot_general(x1, w0[:, 2 * C:], (((0,), (1,)), ((), ())),
                          preferred_element_type=jnp.float32,
                      precision=lax.Precision.HIGHEST)
        + b0_ref[...])
  base_ref[0] = bt

  @pl.when(it == 0)
  def _():
    f2 = feat2_ref[0]
    pcv = (lax.dot_general(f2, w0[:, C:2 * C], (((0,), (1,)), ((), ())),
                           preferred_element_type=jnp.float32,
                      precision=lax.Precision.HIGHEST)
           + lax.dot_general(x2, w0[:, 2 * C:], (((0,), (1,)), ((), ())),
                             preferred_element_type=jnp.float32,
                      precision=lax.Precision.HIGHEST))
    pc_ref[0] = pcv


def _knn_proj_call(xyz1, xyz2, feat1, feat2, w0, b0_2d):
  return pl.pallas_call(
      _knn_proj_body,
      grid=(B, N // TI),
      in_specs=[
          pl.BlockSpec((1, 3, TI), lambda b, it: (b, 0, it)),
          pl.BlockSpec((1, 3, N2), lambda b, it: (b, 0, 0)),
          pl.BlockSpec((1, C, TI), lambda b, it: (b, 0, it)),
          pl.BlockSpec((1, C, N2), lambda b, it: (b, 0, 0)),
          pl.BlockSpec((C, 131), lambda b, it: (0, 0)),
          pl.BlockSpec((1, C), lambda b, it: (0, 0)),
      ],
      out_specs=[
          pl.BlockSpec((1, TI, KNN), lambda b, it: (b, it, 0)),
          pl.BlockSpec((1, TI, C), lambda b, it: (b, it, 0)),
          pl.BlockSpec((1, N2, C), lambda b, it: (b, 0, 0)),
      ],
      out_shape=[
          jax.ShapeDtypeStruct((B, N, KNN), jnp.int32),
          jax.ShapeDtypeStruct((B, N, C), jnp.float32),
          jax.ShapeDtypeStruct((B, N2, C), jnp.float32),
      ],
      scratch_shapes=[pltpu.VMEM((TI, N2), jnp.float32)],
      compiler_params=pltpu.CompilerParams(
          dimension_semantics=("arbitrary", "arbitrary")),
  )(xyz1, xyz2, feat1, feat2, w0, b0_2d)


def _sc_gather_body(table_hbm, idx_hbm, out_hbm, idx_v, rows_v, sem):
  wid = lax.axis_index("s") * 2 + lax.axis_index("c")
  base = wid * ROWS_PER_W

  pltpu.sync_copy(idx_hbm.at[pl.ds(base, ROWS_PER_W)], idx_v)

  def grp(gi, carry):
    cps = []
    for u in range(SC_GRP):
      cps.append(pltpu.async_copy(
          table_hbm.at[idx_v.at[pl.ds((gi * SC_GRP + u) * SC_CHUNK,
                                      SC_CHUNK)]],
          rows_v.at[pl.ds(u * SC_CHUNK, SC_CHUNK)], sem))
    for cp in cps:
      cp.wait()
    pltpu.sync_copy(rows_v, out_hbm.at[pl.ds(base + gi * SC_GRP * SC_CHUNK,
                                             SC_GRP * SC_CHUNK)])
    return carry

  lax.fori_loop(0, SC_NCHUNK // SC_GRP, grp, 0)


@functools.cache
def _get_sc_gather():
  return pl.kernel(
      _sc_gather_body,
      out_type=jax.ShapeDtypeStruct((TOTAL_ROWS, C), jnp.float32),
      mesh=plsc.VectorSubcoreMesh(core_axis_name="c", subcore_axis_name="s"),
      scratch_types=[
          pltpu.VMEM((ROWS_PER_W,), jnp.int32),
          pltpu.VMEM((SC_GRP * SC_CHUNK, C), jnp.float32),
          pltpu.SemaphoreType.DMA,
      ],
      compiler_params=pltpu.CompilerParams(use_tc_tiling_on_sc=False),
  )


def _group_mat():
  i = lax.broadcasted_iota(jnp.int32, (C, C), 0)
  j = lax.broadcasted_iota(jnp.int32, (C, C), 1)
  return ((i // 16) == (j // 16)).astype(jnp.float32)


def _mlp_body(g_ref, base_ref, w1_ref, b1_ref, g0_ref, beta0_ref,
              g1_ref, beta1_ref, out_ref, z_ref):
  rows = N * KNN // 2
  tt = 2048
  qt = tt // (KNN // 2)
  nt = rows // tt
  gm = _group_mat()
  cnt = 16.0 * N * KNN

  def pair(x):
    return jnp.concatenate([x, x], axis=1)

  def tile_y(t):
    gt = g_ref[0, pl.ds(t * tt, tt), :]
    bt = base_ref[0, pl.ds(t * qt, qt), :]
    bp = jnp.concatenate([bt, bt], axis=1)
    y = gt.reshape(qt, KNN // 2, 2 * C) + bp[:, None, :]
    return y.reshape(tt, 2 * C)

  def fold(s):
    return s[:, :C] + s[:, C:]

  def p1(t, carry):
    s, q = carry
    y = tile_y(t)
    return (s + jnp.sum(y, axis=0, keepdims=True),
            q + jnp.sum(y * y, axis=0, keepdims=True))

  z1 = jnp.zeros((1, 2 * C), jnp.float32)
  s0, q0 = lax.fori_loop(0, nt, p1, (z1, z1))
  mean0 = jnp.dot(fold(s0), gm, preferred_element_type=jnp.float32,
                  precision=lax.Precision.HIGHEST) / cnt
  var0 = jnp.dot(fold(q0), gm, preferred_element_type=jnp.float32,
                 precision=lax.Precision.HIGHEST) / cnt - mean0 * mean0
  inv0 = lax.rsqrt(var0 + EPS)
  sc0 = pair(inv0 * g0_ref[...])
  sh0 = pair(beta0_ref[...] - mean0 * inv0 * g0_ref[...])

  w1 = w1_ref[...]
  zc = jnp.zeros((C, C), jnp.float32)
  w2 = jnp.concatenate([jnp.concatenate([w1, zc], axis=1),
                        jnp.concatenate([zc, w1], axis=1)], axis=0)
  b1p = pair(b1_ref[...])

  def p2(t, carry):
    s, q = carry
    ya = tile_y(t) * sc0 + sh0
    ya = jnp.where(ya >= 0, ya, NEG_SLOPE * ya)
    z = lax.dot_general(ya, w2, (((1,), (1,)), ((), ())),
                        preferred_element_type=jnp.float32,
                        precision=lax.Precision.HIGHEST) + b1p
    z_ref[pl.ds(t * tt, tt), :] = z
    return (s + jnp.sum(z, axis=0, keepdims=True),
            q + jnp.sum(z * z, axis=0, keepdims=True))

  s1, q1 = lax.fori_loop(0, nt, p2, (z1, z1))
  mean1 = jnp.dot(fold(s1), gm, preferred_element_type=jnp.float32,
                  precision=lax.Precision.HIGHEST) / cnt
  var1 = jnp.dot(fold(q1), gm, preferred_element_type=jnp.float32,
                 precision=lax.Precision.HIGHEST) / cnt - mean1 * mean1
  inv1 = lax.rsqrt(var1 + EPS)
  sc1 = pair(inv1 * g1_ref[...])
  sh1 = pair(beta1_ref[...] - mean1 * inv1 * g1_ref[...])

  def p3(t, carry):
    z = z_ref[pl.ds(t * tt, tt), :]
    za = z * sc1 + sh1
    za = jnp.where(za >= 0, za, NEG_SLOPE * za)
    zm = jnp.max(za.reshape(qt, KNN // 2, 2 * C), axis=1)
    zq = jnp.maximum(zm[:, :C], zm[:, C:])
    out_ref[0, :, pl.ds(t * qt, qt)] = zq.T
    return carry

  lax.fori_loop(0, nt, p3, 0)


def _mlp_call(g2, base, w1, b1_2d, g0_2d, beta0_2d, g1_2d, beta1_2d):
  return pl.pallas_call(
      _mlp_body,
      grid=(B,),
      in_specs=[
          pl.BlockSpec((1, N * KNN // 2, 2 * C), lambda b: (b, 0, 0)),
          pl.BlockSpec((1, N, C), lambda b: (b, 0, 0)),
          pl.BlockSpec((C, C), lambda b: (0, 0)),
          pl.BlockSpec((1, C), lambda b: (0, 0)),
          pl.BlockSpec((1, C), lambda b: (0, 0)),
          pl.BlockSpec((1, C), lambda b: (0, 0)),
          pl.BlockSpec((1, C), lambda b: (0, 0)),
          pl.BlockSpec((1, C), lambda b: (0, 0)),
      ],
      out_specs=pl.BlockSpec((1, C, N), lambda b: (b, 0, 0)),
      out_shape=jax.ShapeDtypeStruct((B, C, N), jnp.float32),
      scratch_shapes=[pltpu.VMEM((N * KNN // 2, 2 * C), jnp.float32)],
      compiler_params=pltpu.CompilerParams(
          dimension_semantics=("arbitrary",)),
  )(g2, base, w1, b1_2d, g0_2d, beta0_2d, g1_2d, beta1_2d)


def kernel(xyz1, xyz2, feat1, feat2, W0, b0, g0, beta0, W1, b1, g1, beta1):
  idx, base, pc = _knn_proj_call(xyz1, xyz2, feat1, feat2, W0,
                                 b0.reshape(1, C))
  gathered = _get_sc_gather()(pc.reshape(B * N2, C), idx.reshape(TOTAL_ROWS))
  return _mlp_call(gathered.reshape(B, N * KNN // 2, 2 * C), base, W1,
                   b1.reshape(1, C), g0.reshape(1, C), beta0.reshape(1, C),
                   g1.reshape(1, C), beta1.reshape(1, C))


# speedup vs baseline: 16.3579x; 1.0501x over previous
import functools

import jax
import jax.numpy as jnp
from jax import lax
from jax.experimental import pallas as pl
from jax.experimental.pallas import tpu as pltpu
from jax.experimental.pallas import tpu_sc as plsc

KNN = 16
B, N, N2 = 8, 2048, 2048
C = 64
TI = 256
TT = 4096
EPS = 1e-5
NEG_SLOPE = 0.1

SC_WORKERS = 32
TOTAL_ROWS = B * N * KNN
ROWS_PER_W = TOTAL_ROWS // SC_WORKERS
SC_CHUNK = 128
SC_NCHUNK = ROWS_PER_W // SC_CHUNK
SC_GRP = 8


def _knn_proj_body(xyz1_ref, xyz2_ref, feat1_ref, feat2_ref, w0_ref, b0_ref,
                   idx_ref, base_ref, pc_ref, s_ref):
  b = pl.program_id(0)
  it = pl.program_id(1)
  x1 = xyz1_ref[0]
  x2 = xyz2_ref[0]
  w0 = w0_ref[...]

  n2 = jnp.sum(x2 * x2, axis=0, keepdims=True)
  g = lax.dot_general(x1, x2, (((0,), (0,)), ((), ())),
                      preferred_element_type=jnp.float32,
                      precision=lax.Precision.HIGHEST)
  s_ref[...] = n2 - 2.0 * g

  lane = lax.broadcasted_iota(jnp.int32, (8, 128), 1)
  nvr = N2 // 128
  inf = jnp.float32(jnp.inf)

  def topk8(sb):
    m = functools.reduce(jnp.minimum, sb)
    cols = []
    for r in range(KNN):
      gv = jnp.min(m, axis=1, keepdims=True)
      mparts, jparts = [], []
      for g0 in range(0, nvr, 4):
        mp, jp = [], []
        for v in range(g0, g0 + 4):
          emv = sb[v] == gv
          jp.append(jnp.where(emv, lane + v * 128, N2))
          if r < KNN - 1:
            sb[v] = jnp.where(emv, inf, sb[v])
            mp.append(sb[v])
        jparts.append(jnp.minimum(jnp.minimum(jp[0], jp[1]),
                                  jnp.minimum(jp[2], jp[3])))
        if r < KNN - 1:
          mparts.append(jnp.minimum(jnp.minimum(mp[0], mp[1]),
                                    jnp.minimum(mp[2], mp[3])))
      if r < KNN - 1:
        m = jnp.minimum(jnp.minimum(mparts[0], mparts[1]),
                        jnp.minimum(mparts[2], mparts[3]))
      gj = functools.reduce(jnp.minimum, jparts)
      cols.append(jnp.min(gj, axis=1, keepdims=True))
    return jnp.concatenate(cols, axis=1)

  def blk(j, carry):
    for u in range(32):
      row = j * 256 + u * 8
      sbw = s_ref[pl.ds(row, 8), :]
      sb = [sbw[:, v * 128:(v + 1) * 128] for v in range(nvr)]
      idx_ref[0, pl.ds(row, 8), :] = topk8(sb) + b * N2
    return carry

  lax.fori_loop(0, TI // 256, blk, 0)

  f1 = feat1_ref[0]
  bt = (lax.dot_general(f1, w0[:, :C], (((0,), (1,)), ((), ())),
                        preferred_element_type=jnp.float32,
                      precision=lax.Precision.HIGHEST)
        - lax.dot_general(x1, w0[:, 2 * C:], (((0,), (1,)), ((), ())),
                          preferred_element_type=jnp.float32,
                      precision=lax.Precision.HIGHEST)
        + b0_ref[...])
  base_ref[0] = bt

  @pl.when(it == 0)
  def _():
    f2 = feat2_ref[0]
    pcv = (lax.dot_general(f2, w0[:, C:2 * C], (((0,), (1,)), ((), ())),
                           preferred_element_type=jnp.float32,
                      precision=lax.Precision.HIGHEST)
           + lax.dot_general(x2, w0[:, 2 * C:], (((0,), (1,)), ((), ())),
                             preferred_element_type=jnp.float32,
                      precision=lax.Precision.HIGHEST))
    pc_ref[0] = pcv


def _knn_proj_call(xyz1, xyz2, feat1, feat2, w0, b0_2d):
  return pl.pallas_call(
      _knn_proj_body,
      grid=(B, N // TI),
      in_specs=[
          pl.BlockSpec((1, 3, TI), lambda b, it: (b, 0, it)),
          pl.BlockSpec((1, 3, N2), lambda b, it: (b, 0, 0)),
          pl.BlockSpec((1, C, TI), lambda b, it: (b, 0, it)),
          pl.BlockSpec((1, C, N2), lambda b, it: (b, 0, 0)),
          pl.BlockSpec((C, 131), lambda b, it: (0, 0)),
          pl.BlockSpec((1, C), lambda b, it: (0, 0)),
      ],
      out_specs=[
          pl.BlockSpec((1, TI, KNN), lambda b, it: (b, it, 0)),
          pl.BlockSpec((1, TI, C), lambda b, it: (b, it, 0)),
          pl.BlockSpec((1, N2, C), lambda b, it: (b, 0, 0)),
      ],
      out_shape=[
          jax.ShapeDtypeStruct((B, N, KNN), jnp.int32),
          jax.ShapeDtypeStruct((B, N, C), jnp.float32),
          jax.ShapeDtypeStruct((B, N2, C), jnp.float32),
      ],
      scratch_shapes=[pltpu.VMEM((TI, N2), jnp.float32)],
      compiler_params=pltpu.CompilerParams(
          dimension_semantics=("arbitrary", "arbitrary")),
  )(xyz1, xyz2, feat1, feat2, w0, b0_2d)


def _sc_gather_body(table_hbm, idx_hbm, out_hbm, idx_v, rows_v, sem):
  wid = lax.axis_index("s") * 2 + lax.axis_index("c")
  base = wid * ROWS_PER_W

  pltpu.sync_copy(idx_hbm.at[pl.ds(base, ROWS_PER_W)], idx_v)

  def grp(gi, carry):
    cps = []
    for u in range(SC_GRP):
      cps.append(pltpu.async_copy(
          table_hbm.at[idx_v.at[pl.ds((gi * SC_GRP + u) * SC_CHUNK,
                                      SC_CHUNK)]],
          rows_v.at[pl.ds(u * SC_CHUNK, SC_CHUNK)], sem))
    for cp in cps:
      cp.wait()
    pltpu.sync_copy(rows_v, out_hbm.at[pl.ds(base + gi * SC_GRP * SC_CHUNK,
                                             SC_GRP * SC_CHUNK)])
    return carry

  lax.fori_loop(0, SC_NCHUNK // SC_GRP, grp, 0)


@functools.cache
def _get_sc_gather():
  return pl.kernel(
      _sc_gather_body,
      out_type=jax.ShapeDtypeStruct((TOTAL_ROWS, C), jnp.float32),
      mesh=plsc.VectorSubcoreMesh(core_axis_name="c", subcore_axis_name="s"),
      scratch_types=[
          pltpu.VMEM((ROWS_PER_W,), jnp.int32),
          pltpu.VMEM((SC_GRP * SC_CHUNK, C), jnp.float32),
          pltpu.SemaphoreType.DMA,
      ],
      compiler_params=pltpu.CompilerParams(use_tc_tiling_on_sc=False),
  )


def _group_mat():
  i = lax.broadcasted_iota(jnp.int32, (C, C), 0)
  j = lax.broadcasted_iota(jnp.int32, (C, C), 1)
  return ((i // 16) == (j // 16)).astype(jnp.float32)


def _mlp_body(g_ref, base_ref, w1_ref, b1_ref, g0_ref, beta0_ref,
              g1_ref, beta1_ref, out_ref, z_ref):
  rows = N * KNN // 2
  tt = 2048
  qt = tt // (KNN // 2)
  nt = rows // tt
  gm = _group_mat()
  cnt = 16.0 * N * KNN

  def pair(x):
    return jnp.concatenate([x, x], axis=1)

  def tile_y(t):
    gt = g_ref[0, pl.ds(t * tt, tt), :]
    bt = base_ref[0, pl.ds(t * qt, qt), :]
    bp = jnp.concatenate([bt, bt], axis=1)
    y = gt.reshape(qt, KNN // 2, 2 * C) + bp[:, None, :]
    return y.reshape(tt, 2 * C)

  def fold(s):
    return s[:, :C] + s[:, C:]

  def p1(t, carry):
    s, q = carry
    y = tile_y(t)
    return (s + jnp.sum(y, axis=0, keepdims=True),
            q + jnp.sum(y * y, axis=0, keepdims=True))

  z1 = jnp.zeros((1, 2 * C), jnp.float32)
  s0, q0 = lax.fori_loop(0, nt, p1, (z1, z1))
  mean0 = jnp.dot(fold(s0), gm, preferred_element_type=jnp.float32,
                  precision=lax.Precision.HIGHEST) / cnt
  var0 = jnp.dot(fold(q0), gm, preferred_element_type=jnp.float32,
                 precision=lax.Precision.HIGHEST) / cnt - mean0 * mean0
  inv0 = lax.rsqrt(var0 + EPS)
  sc0 = pair(inv0 * g0_ref[...])
  sh0 = pair(beta0_ref[...] - mean0 * inv0 * g0_ref[...])

  w1 = w1_ref[...]
  zc = jnp.zeros((C, C), jnp.float32)
  w2 = jnp.concatenate([jnp.concatenate([w1, zc], axis=1),
                        jnp.concatenate([zc, w1], axis=1)], axis=0)
  b1p = pair(b1_ref[...])

  def p2(t, carry):
    s, q = carry
    ya = tile_y(t) * sc0 + sh0
    ya = jnp.where(ya >= 0, ya, NEG_SLOPE * ya)
    z = lax.dot_general(ya, w2, (((1,), (1,)), ((), ())),
                        preferred_element_type=jnp.float32,
                        precision=lax.Precision.HIGHEST) + b1p
    z_ref[pl.ds(t * tt, tt), :] = z
    return (s + jnp.sum(z, axis=0, keepdims=True),
            q + jnp.sum(z * z, axis=0, keepdims=True))

  s1, q1 = lax.fori_loop(0, nt, p2, (z1, z1))
  mean1 = jnp.dot(fold(s1), gm, preferred_element_type=jnp.float32,
                  precision=lax.Precision.HIGHEST) / cnt
  var1 = jnp.dot(fold(q1), gm, preferred_element_type=jnp.float32,
                 precision=lax.Precision.HIGHEST) / cnt - mean1 * mean1
  inv1 = lax.rsqrt(var1 + EPS)
  sc1 = pair(inv1 * g1_ref[...])
  sh1 = pair(beta1_ref[...] - mean1 * inv1 * g1_ref[...])

  def p3(t, carry):
    z = z_ref[pl.ds(t * tt, tt), :]
    za = z * sc1 + sh1
    za = jnp.where(za >= 0, za, NEG_SLOPE * za)
    zm = jnp.max(za.reshape(qt, KNN // 2, 2 * C), axis=1)
    zq = jnp.maximum(zm[:, :C], zm[:, C:])
    out_ref[0, :, pl.ds(t * qt, qt)] = zq.T
    return carry

  lax.fori_loop(0, nt, p3, 0)


def _mlp_call(g2, base, w1, b1_2d, g0_2d, beta0_2d, g1_2d, beta1_2d):
  return pl.pallas_call(
      _mlp_body,
      grid=(B,),
      in_specs=[
          pl.BlockSpec((1, N * KNN // 2, 2 * C), lambda b: (b, 0, 0)),
          pl.BlockSpec((1, N, C), lambda b: (b, 0, 0)),
          pl.BlockSpec((C, C), lambda b: (0, 0)),
          pl.BlockSpec((1, C), lambda b: (0, 0)),
          pl.BlockSpec((1, C), lambda b: (0, 0)),
          pl.BlockSpec((1, C), lambda b: (0, 0)),
          pl.BlockSpec((1, C), lambda b: (0, 0)),
          pl.BlockSpec((1, C), lambda b: (0, 0)),
      ],
      out_specs=pl.BlockSpec((1, C, N), lambda b: (b, 0, 0)),
      out_shape=jax.ShapeDtypeStruct((B, C, N), jnp.float32),
      scratch_shapes=[pltpu.VMEM((N * KNN // 2, 2 * C), jnp.float32)],
      compiler_params=pltpu.CompilerParams(
          dimension_semantics=("arbitrary",)),
  )(g2, base, w1, b1_2d, g0_2d, beta0_2d, g1_2d, beta1_2d)


def kernel(xyz1, xyz2, feat1, feat2, W0, b0, g0, beta0, W1, b1, g1, beta1):
  idx, base, pc = _knn_proj_call(xyz1, xyz2, feat1, feat2, W0,
                                 b0.reshape(1, C))
  gathered = _get_sc_gather()(pc.reshape(B * N2, C), idx.reshape(TOTAL_ROWS))
  return _mlp_call(gathered.reshape(B, N * KNN // 2, 2 * C), base, W1,
                   b1.reshape(1, C), g0.reshape(1, C), beta0.reshape(1, C),
                   g1.reshape(1, C), beta1.reshape(1, C))
